# baseline (speedup 1.0000x reference)
import jax
import jax.numpy as jnp
from jax import lax
from jax.experimental import pallas as pl
from jax.experimental.pallas import tpu as pltpu
from jax.experimental.pallas import tpu_sc as plsc

N = 10000
E = 320000
H = 128
L = 3
HL = (L * H) // 2
BLK = 1000


def _full(shape):
    return pl.BlockSpec(shape, lambda i: tuple(0 for _ in shape))


def _rows(shape):
    return pl.BlockSpec(shape, lambda i: (i,) + tuple(0 for _ in shape[1:]))


def _enc_body(x_ref, w_ref, a2_ref, h_ref, s_ref):
    h = jnp.dot(x_ref[...], w_ref[...], preferred_element_type=jnp.float32)
    h_ref[...] = h
    s_ref[...] = jnp.dot(h, a2_ref[...], preferred_element_type=jnp.float32)


def _tc_enc(x, w, a2):
    return pl.pallas_call(
        _enc_body,
        grid=(N // BLK,),
        in_specs=[_rows((BLK, H)), _full((H, H)), _full((H, 2))],
        out_specs=[_rows((BLK, H)), _rows((BLK, 2))],
        out_shape=[
            jax.ShapeDtypeStruct((N, H), jnp.float32),
            jax.ShapeDtypeStruct((N, 2), jnp.float32),
        ],
    )(x, w, a2)


def _cat_relu(p0, p1, b):
    z = jnp.concatenate([p0[0][...] + p1[0][...],
                         p0[1][...] + p1[1][...]], axis=1)
    return jnp.maximum(z + b, 0.0)


def _fuse_body(p00_ref, p01_ref, p10_ref, p11_ref, b_ref, w_ref, a2_ref,
               z_ref, h_ref, s_ref):
    z = _cat_relu((p00_ref, p01_ref), (p10_ref, p11_ref), b_ref[...])
    z_ref[...] = z
    h = jnp.dot(z, w_ref[...], preferred_element_type=jnp.float32)
    h_ref[...] = h
    s_ref[...] = jnp.dot(h, a2_ref[...], preferred_element_type=jnp.float32)


def _tc_fuse(p0, p1, b, w, a2):
    return pl.pallas_call(
        _fuse_body,
        grid=(N // BLK,),
        in_specs=[_rows((BLK, H // 2)), _rows((BLK, H // 2)),
                  _rows((BLK, H // 2)), _rows((BLK, H // 2)), _full((1, H)),
                  _full((H, H)), _full((H, 2))],
        out_specs=[_rows((BLK, H)), _rows((BLK, H)), _rows((BLK, 2))],
        out_shape=[
            jax.ShapeDtypeStruct((N, H), jnp.float32),
            jax.ShapeDtypeStruct((N, H), jnp.float32),
            jax.ShapeDtypeStruct((N, 2), jnp.float32),
        ],
    )(p0[0], p0[1], p1[0], p1[1], b.reshape(1, H), w, a2)


def _lstm_steps(zs, h0, c0, w_ih, w_hh, bsum, order):
    h, c = h0, c0
    outs = [None, None, None]
    for t in order:
        g = (jnp.dot(zs[t], w_ih, preferred_element_type=jnp.float32)
             + jnp.dot(h, w_hh, preferred_element_type=jnp.float32) + bsum)
        i = jax.nn.sigmoid(g[:, 0 * HL:1 * HL])
        f = jax.nn.sigmoid(g[:, 1 * HL:2 * HL])
        gg = jnp.tanh(g[:, 2 * HL:3 * HL])
        o = jax.nn.sigmoid(g[:, 3 * HL:4 * HL])
        c = f * c + i * gg
        h = o * jnp.tanh(c)
        outs[t] = h
    return outs


def _jk_body(p00_ref, p01_ref, p10_ref, p11_ref, b_ref, z1_ref, z2_ref,
             wihf_ref, whhf_ref, bf_ref, wihb_ref, whhb_ref, bb_ref,
             attw_ref, w1_ref, b1_ref, w2_ref, b2_ref,
             weca_ref, wecb_ref, ecb1_ref,
             node_ref, p_ref, q_ref):
    z3 = _cat_relu((p00_ref, p01_ref), (p10_ref, p11_ref), b_ref[...])
    zs = [z1_ref[...], z2_ref[...], z3]
    zero = jnp.zeros((zs[0].shape[0], HL), jnp.float32)
    of = _lstm_steps(zs, zero, zero, wihf_ref[...], whhf_ref[...],
                     bf_ref[...], (0, 1, 2))
    ob = _lstm_steps(zs, zero, zero, wihb_ref[...], whhb_ref[...],
                     bb_ref[...], (2, 1, 0))
    attw = attw_ref[...]
    logits = jnp.concatenate(
        [jnp.dot(jnp.concatenate([of[t], ob[t]], axis=1), attw,
                 preferred_element_type=jnp.float32) for t in range(3)],
        axis=1)
    alpha = jax.nn.softmax(logits, axis=1)
    zj = (alpha[:, 0:1] * zs[0] + alpha[:, 1:2] * zs[1]
          + alpha[:, 2:3] * zs[2])
    r = jnp.maximum(jnp.dot(zj, w1_ref[...],
                            preferred_element_type=jnp.float32) + b1_ref[...],
                    0.0)
    nl = jnp.dot(r, w2_ref[...], preferred_element_type=jnp.float32) + b2_ref[...]
    node_ref[...] = jax.nn.softmax(nl, axis=1)
    p_ref[...] = (jnp.dot(zj, weca_ref[...], preferred_element_type=jnp.float32)
                  + ecb1_ref[...])
    q_ref[...] = jnp.dot(zj, wecb_ref[...], preferred_element_type=jnp.float32)


def _tc_jk(p0, p1, b, z1, z2, jk, nc, weca, wecb, ecb1):
    bf = (jk['b_ih_f'] + jk['b_hh_f']).reshape(1, 4 * HL)
    bb = (jk['b_ih_b'] + jk['b_hh_b']).reshape(1, 4 * HL)
    return pl.pallas_call(
        _jk_body,
        grid=(N // BLK,),
        in_specs=[_rows((BLK, H // 2)), _rows((BLK, H // 2)),
                  _rows((BLK, H // 2)), _rows((BLK, H // 2)), _full((1, H)),
                  _rows((BLK, H)), _rows((BLK, H)),
                  _full((H, 4 * HL)), _full((HL, 4 * HL)), _full((1, 4 * HL)),
                  _full((H, 4 * HL)), _full((HL, 4 * HL)), _full((1, 4 * HL)),
                  _full((2 * HL, 1)),
                  _full((H, H)), _full((1, H)), _full((H, 2)), _full((1, 2)),
                  _full((H, H)), _full((H, H)), _full((1, H))],
        out_specs=[_rows((BLK, 2)), _rows((BLK, H)), _rows((BLK, H))],
        out_shape=[
            jax.ShapeDtypeStruct((N, 2), jnp.float32),
            jax.ShapeDtypeStruct((N, H), jnp.float32),
            jax.ShapeDtypeStruct((N, H), jnp.float32),
        ],
    )(p0[0], p0[1], p1[0], p1[1], b.reshape(1, H),
      z1, z2,
      jk['W_ih_f'].T, jk['W_hh_f'].T, bf,
      jk['W_ih_b'].T, jk['W_hh_b'].T, bb,
      jk['att_W'].reshape(2 * HL, 1),
      nc['W1'], nc['b1'].reshape(1, H), nc['W2'], nc['b2'].reshape(1, 2),
      weca, wecb, ecb1.reshape(1, H))


_NW = 32
_EPW = E // _NW
_CA = 2000
_CR = 80
_NCH = _EPW // _CR
_NP = 10240
_RPS = _NP // 16


def _sc_gat_body(has_ea, args):
    if has_ea:
        (h2_hbm, s_hbm, nbr3d_h, agg3d_h, ea3d_h,
         o0_hbm, o1_hbm,
         s_tab, nbr_c, agg_c, ea_c, nbr_c1, agg_c1, ea_c1,
         e_arr, nbr2d, agg2d, stab, rows, rows1,
         rowidx, idxg, idxg1, sh_ssum, sh_out,
         sem_g0, sem_s0, sem_g1, sem_s1) = args
    else:
        (h2_hbm, s_hbm, nbr3d_h, agg3d_h,
         o0_hbm, o1_hbm,
         s_tab, nbr_c, agg_c, ea_c, nbr_c1, agg_c1, ea_c1,
         e_arr, nbr2d, agg2d, stab, rows, rows1,
         rowidx, idxg, idxg1, sh_ssum, sh_out,
         sem_g0, sem_s0, sem_g1, sem_s1) = args
        ea3d_h = None
    ci = lax.axis_index("c")
    si = lax.axis_index("s")
    wid = si * 2 + ci
    zero16 = jnp.zeros((16,), jnp.float32)
    iota16 = lax.iota(jnp.int32, 16)

    def _z80(i, _):
        for kk in range(8):
            stab[i, pl.ds(kk * 16, 16)] = zero16
        return 0
    lax.fori_loop(0, 80, _z80, 0)

    def _zrows(i, _):
        for kk in range(4):
            rows[i, pl.ds(kk * 16, 16)] = zero16
        return 0
    lax.fori_loop(0, _CR, _zrows, 0)

    @pl.when(si == 0)
    def _():
        pltpu.sync_copy(stab, sh_ssum)

    for g in range(5):
        rowidx[pl.ds(g * 16, 16)] = iota16 + g * 16

    pltpu.sync_copy(s_hbm, s_tab)
    pltpu.sync_copy(nbr3d_h.at[wid], nbr2d)
    pltpu.sync_copy(agg3d_h.at[wid], agg2d)
    plsc.subcore_barrier()

    rpc = _CA // _CR
    npc = _EPW // _CA
    other = si * 2 + (1 - ci)
    pa_bufs = ((nbr_c, agg_c, ea_c, sem_g0), (nbr_c1, agg_c1, ea_c1, sem_g1))

    def _pa_issue(t, b):
        nb_b, ag_b, ea_b, sg = pa_bufs[b]
        sl = jnp.where(t < npc, wid, other)
        ch = lax.rem(t, npc)
        pltpu.async_copy(nbr3d_h.at[sl, pl.ds(ch * rpc, rpc)], nb_b, sg)
        pltpu.async_copy(agg3d_h.at[sl, pl.ds(ch * rpc, rpc)], ag_b, sg)
        if ea3d_h is not None:
            pltpu.async_copy(ea3d_h.at[sl, pl.ds(ch * rpc, rpc)], ea_b, sg)

    def _pa_wait(t, b):
        nb_b, ag_b, ea_b, sg = pa_bufs[b]
        sl = jnp.where(t < npc, wid, other)
        ch = lax.rem(t, npc)
        pltpu.make_async_copy(nbr3d_h.at[sl, pl.ds(ch * rpc, rpc)],
                              nb_b, sg).wait()
        pltpu.make_async_copy(agg3d_h.at[sl, pl.ds(ch * rpc, rpc)],
                              ag_b, sg).wait()
        if ea3d_h is not None:
            pltpu.make_async_copy(ea3d_h.at[sl, pl.ds(ch * rpc, rpc)],
                                  ea_b, sg).wait()

    def _pa_process(t, b):
        nb_b, ag_b, ea_b, sg = pa_bufs[b]
        ch = lax.rem(t, npc)

        def _r25(r, _):
            for g in range(_CR // 16):
                nb = nb_b[r, pl.ds(g * 16, 16)]
                ag = ag_b[r, pl.ds(g * 16, 16)]
                sa = plsc.load_gather(s_tab, [nb * 2])
                sb = plsc.load_gather(s_tab, [ag * 2 + 1])
                al = sa + sb
                if ea3d_h is not None:
                    al = al + ea_b[r, pl.ds(g * 16, 16)]
                al = jnp.where(al >= 0.0, al, al * 0.2)
                ev = jnp.exp(al)

                @pl.when(t < npc)
                def _(ev=ev, ch=ch, r=r, g=g):
                    e_arr[pl.ds(ch * _CA + r * _CR + g * 16, 16)] = ev
                rr = lax.shift_right_logical(ag, 7)
                cc = jnp.bitwise_and(ag, 127)
                plsc.addupdate_scatter(stab, [rr, cc], ev)
            return 0
        lax.fori_loop(0, rpc, _r25, 0)

    _pa_issue(0, 0)

    def _pa_pair(i, _):
        t0 = 2 * i
        _pa_issue(t0 + 1, 1)
        _pa_wait(t0, 0)
        _pa_process(t0, 0)

        @pl.when(t0 + 2 < 2 * npc)
        def _(t0=t0):
            _pa_issue(t0 + 2, 0)
        _pa_wait(t0 + 1, 1)
        _pa_process(t0 + 1, 1)
        return 0
    lax.fori_loop(0, npc, _pa_pair, 0)

    pltpu.sync_copy(stab, sh_ssum.at[rowidx], add=True)
    plsc.subcore_barrier()
    pltpu.sync_copy(sh_ssum, stab)

    def _wchunk(ch, _):
        for g in range(_CR // 16):
            ag = agg2d[ch, pl.ds(g * 16, 16)]
            rr = lax.shift_right_logical(ag, 7)
            cc = jnp.bitwise_and(ag, 127)
            ssum = plsc.load_gather(stab, [rr, cc])
            ev = e_arr[pl.ds(ch * _CR + g * 16, 16)]
            e_arr[pl.ds(ch * _CR + g * 16, 16)] = ev / (ssum + 1e-16)
        return 0
    lax.fori_loop(0, _NCH, _wchunk, 0)

    bufs = ((rows, idxg, sem_g0, sem_s0), (rows1, idxg1, sem_g1, sem_s1))

    def _build_idx(ch, idx_ref, half):
        for g in range(_CR // 16):
            nb = nbr2d[ch, pl.ds(g * 16, 16)]
            idx_ref[pl.ds(g * 16, 16)] = nb * 2 + half

    def _scale(ch, rows_ref):
        def _row5(i, _):
            j0 = i * 5
            wv = e_arr[pl.ds(ch * _CR + j0, 16)]
            for r5 in range(5):
                ws = wv[r5]
                for kk in range(4):
                    rows_ref[j0 + r5, pl.ds(kk * 16, 16)] = (
                        rows_ref[j0 + r5, pl.ds(kk * 16, 16)] * ws)
            return 0
        lax.fori_loop(0, _CR // 5, _row5, 0)

    for half in range(2):
        lax.fori_loop(0, _CR, _zrows, 0)
        for r in range(_RPS // _CR):
            pltpu.sync_copy(rows, sh_out.at[pl.ds(si * _RPS + r * _CR, _CR)])
        plsc.subcore_barrier()

        def _pair(i, _, half=half):
            for b, (rows_b, idx_b, sg, ss) in enumerate(bufs):
                ch = 2 * i + b

                @pl.when(i > 0)
                def _(rows_b=rows_b, ss=ss):
                    pltpu.make_async_copy(
                        rows_b, sh_out.at[agg2d.at[0]], ss).wait()
                _build_idx(ch, idx_b, half)
                pltpu.async_copy(h2_hbm.at[idx_b], rows_b, sg)
            for b, (rows_b, idx_b, sg, ss) in enumerate(bufs):
                ch = 2 * i + b
                pltpu.make_async_copy(h2_hbm.at[idx_b], rows_b, sg).wait()
                _scale(ch, rows_b)
                pltpu.async_copy(rows_b, sh_out.at[agg2d.at[ch]], ss,
                                 add=True)
            return 0
        lax.fori_loop(0, (_NCH - 1) // 2, _pair, 0)

        for rows_b, idx_b, sg, ss in bufs:
            pltpu.make_async_copy(rows_b, sh_out.at[agg2d.at[0]], ss).wait()
        last = _NCH - 1
        _build_idx(last, idxg, half)
        cp = pltpu.async_copy(h2_hbm.at[idxg], rows, sem_g0)
        cp.wait()
        _scale(last, rows)
        pltpu.sync_copy(rows, sh_out.at[agg2d.at[last]], add=True)
        plsc.subcore_barrier()

        @pl.when(ci == 0)
        def _(half=half):
            pltpu.sync_copy(sh_out.at[pl.ds(si * _RPS, _RPS)],
                            o0_hbm.at[half, pl.ds(si * _RPS, _RPS)])

        @pl.when(ci == 1)
        def _(half=half):
            pltpu.sync_copy(sh_out.at[pl.ds(si * _RPS, _RPS)],
                            o1_hbm.at[half, pl.ds(si * _RPS, _RPS)])
        plsc.subcore_barrier()


def _sc_gat(h, s2, nbr3d, agg3d, eaf=None):
    has_ea = eaf is not None
    mesh = plsc.VectorSubcoreMesh(core_axis_name="c", subcore_axis_name="s")
    scratch = [
        pltpu.VMEM((2 * N,), jnp.float32),
        pltpu.VMEM((_CA // _CR, _CR), jnp.int32),
        pltpu.VMEM((_CA // _CR, _CR), jnp.int32),
        pltpu.VMEM((_CA // _CR, _CR), jnp.float32),
        pltpu.VMEM((_CA // _CR, _CR), jnp.int32),
        pltpu.VMEM((_CA // _CR, _CR), jnp.int32),
        pltpu.VMEM((_CA // _CR, _CR), jnp.float32),
        pltpu.VMEM((_EPW + 16,), jnp.float32),
        pltpu.VMEM((_NCH, _CR), jnp.int32),
        pltpu.VMEM((_NCH, _CR), jnp.int32),
        pltpu.VMEM((80, 128), jnp.float32),
        pltpu.VMEM((_CR, H // 2), jnp.float32),
        pltpu.VMEM((_CR, H // 2), jnp.float32),
        pltpu.VMEM((80,), jnp.int32),
        pltpu.VMEM((_CR,), jnp.int32),
        pltpu.VMEM((_CR,), jnp.int32),
        pltpu.VMEM_SHARED((80, 128), jnp.float32),
        pltpu.VMEM_SHARED((_NP, H // 2), jnp.float32),
        pltpu.SemaphoreType.DMA,
        pltpu.SemaphoreType.DMA,
        pltpu.SemaphoreType.DMA,
        pltpu.SemaphoreType.DMA,
    ]
    out_type = [pltpu.HBM((2, _NP, H // 2), jnp.float32),
                pltpu.HBM((2, _NP, H // 2), jnp.float32)]
    fn = pl.kernel(lambda *args: _sc_gat_body(has_ea, args),
                   out_type=out_type, mesh=mesh, scratch_types=scratch,
                   compiler_params=pltpu.CompilerParams(
                       use_tc_tiling_on_sc=False,
                       needs_layout_passes=False))
    h2 = h.reshape(2 * N, H // 2)
    if has_ea:
        o0, o1 = fn(h2, s2.reshape(-1), nbr3d, agg3d,
                    eaf.reshape(_NW, _NCH, _CR))
    else:
        o0, o1 = fn(h2, s2.reshape(-1), nbr3d, agg3d)
    return ((o0[0, :N], o0[1, :N]), (o1[0, :N], o1[1, :N]))


def _ea_body(attr_ref, we_ref, ae_ref, out_ref):
    w4 = jnp.dot(we_ref[...], ae_ref[...], preferred_element_type=jnp.float32)
    out_ref[...] = jnp.dot(attr_ref[...], w4,
                           preferred_element_type=jnp.float32)


def _tc_ea(edge_attr, we, a_edge):
    eb = E // 16
    out = pl.pallas_call(
        _ea_body,
        grid=(16,),
        in_specs=[_rows((eb, 4)), _full((4, H)), _full((H, 1))],
        out_specs=_rows((eb, 1)),
        out_shape=jax.ShapeDtypeStruct((E, 1), jnp.float32),
    )(edge_attr, we, a_edge.reshape(H, 1))
    return out.reshape(E)


def _sc_ec_body(args):
    (p_hbm, q_hbm, src3d_h, dst3d_h, attr2d_h, w1c_hbm, w2d_hbm,
     d_hbm,
     sidx, didx, attr_c, prow, qrow, prow1, qrow1, d_arr, w1c_v, w2d_v,
     sem_p, sem_q, sem_p1, sem_q1) = args
    ci = lax.axis_index("c")
    si = lax.axis_index("s")
    wid = si * 2 + ci
    zero16 = jnp.zeros((16,), jnp.float32)
    iota16 = lax.iota(jnp.int32, 16)
    m0 = iota16 == 0
    pltpu.sync_copy(src3d_h.at[wid], sidx)
    pltpu.sync_copy(dst3d_h.at[wid], didx)
    pltpu.sync_copy(w1c_hbm, w1c_v)
    pltpu.sync_copy(w2d_hbm, w2d_v)
    bufs = ((prow, qrow, sem_p, sem_q), (prow1, qrow1, sem_p1, sem_q1))

    def _issue(ch, b):
        pr, qr, sp, sq = bufs[b]
        pltpu.async_copy(p_hbm.at[sidx.at[ch]], pr, sp)
        pltpu.async_copy(q_hbm.at[didx.at[ch]], qr, sq)

    def _process(ch, b):
        pr, qr, sp, sq = bufs[b]
        pltpu.sync_copy(attr2d_h.at[wid, pl.ds(ch * 4 * _CR, 4 * _CR)],
                        attr_c.at[pl.ds(0, 4 * _CR)])
        pltpu.make_async_copy(p_hbm.at[sidx.at[ch]], pr, sp).wait()
        pltpu.make_async_copy(q_hbm.at[didx.at[ch]], qr, sq).wait()
        ws0 = tuple(w1c_v[k, pl.ds(kk * 16, 16)]
                    for k in range(4) for kk in range(8))
        ws1 = tuple(w2d_v[pl.ds(kk * 16, 16)] for kk in range(8))

        def _row4(i, carry):
            w1c_r, w2d_r = carry
            j0 = 4 * i
            av = attr_c[pl.ds(4 * j0, 16)]
            for r4 in range(4):
                j = j0 + r4
                o = 4 * r4
                dacc = zero16
                for kk in range(8):
                    u = pr[j, pl.ds(kk * 16, 16)] + qr[j, pl.ds(kk * 16, 16)]
                    u = (u + av[o] * w1c_r[kk] + av[o + 1] * w1c_r[8 + kk]
                         + av[o + 2] * w1c_r[16 + kk]
                         + av[o + 3] * w1c_r[24 + kk])
                    u = jnp.maximum(u, 0.0)
                    dacc = dacc + u * w2d_r[kk]
                dj = jnp.sum(dacc)
                plsc.store_scatter(d_arr, [iota16 * 0 + (ch * _CR + j)],
                                   zero16 + dj, mask=m0)
            return carry
        lax.fori_loop(0, _CR // 4, _row4, (ws0, ws1))

    _issue(0, 0)

    def _pair(i, _):
        _issue(2 * i + 1, 1)
        _process(2 * i, 0)
        _issue(2 * i + 2, 0)
        _process(2 * i + 1, 1)
        return 0
    lax.fori_loop(0, (_NCH - 1) // 2, _pair, 0)
    _process(_NCH - 1, 0)
    pltpu.sync_copy(d_arr, d_hbm.at[pl.ds(wid * _EPW, _EPW)])


def _sc_ec(p, q, src3d, dst3d, attr2d, w1c, w2d):
    mesh = plsc.VectorSubcoreMesh(core_axis_name="c", subcore_axis_name="s")
    scratch = [
        pltpu.VMEM((_NCH, _CR), jnp.int32),
        pltpu.VMEM((_NCH, _CR), jnp.int32),
        pltpu.VMEM((4 * _CR + 16,), jnp.float32),
        pltpu.VMEM((_CR, H), jnp.float32),
        pltpu.VMEM((_CR, H), jnp.float32),
        pltpu.VMEM((_CR, H), jnp.float32),
        pltpu.VMEM((_CR, H), jnp.float32),
        pltpu.VMEM((_EPW,), jnp.float32),
        pltpu.VMEM((4, H), jnp.float32),
        pltpu.VMEM((H,), jnp.float32),
        pltpu.SemaphoreType.DMA,
        pltpu.SemaphoreType.DMA,
        pltpu.SemaphoreType.DMA,
        pltpu.SemaphoreType.DMA,
    ]
    fn = pl.kernel(lambda *args: _sc_ec_body(args),
                   out_type=[pltpu.HBM((E,), jnp.float32)],
                   mesh=mesh, scratch_types=scratch,
                   compiler_params=pltpu.CompilerParams(
                       use_tc_tiling_on_sc=False,
                       needs_layout_passes=False))
    (d,) = fn(p, q, src3d, dst3d, attr2d, w1c, w2d)
    return d


def _ecs_body(d_ref, b2d_ref, o0_ref, o1_ref):
    dd = d_ref[...] + b2d_ref[...]
    o0 = 1.0 / (1.0 + jnp.exp(dd))
    o0_ref[...] = o0
    o1_ref[...] = 1.0 - o0


def _tc_ecs(d, b2d):
    rows = E // 128
    o0, o1 = pl.pallas_call(
        _ecs_body,
        grid=(1,),
        in_specs=[_full((rows, 128)), _full((1, 1))],
        out_specs=[_full((rows, 128)), _full((rows, 128))],
        out_shape=[jax.ShapeDtypeStruct((rows, 128), jnp.float32),
                   jax.ShapeDtypeStruct((rows, 128), jnp.float32)],
    )(d.reshape(rows, 128), b2d.reshape(1, 1))
    return jnp.stack([o0, o1], axis=-1).reshape(E, 2)


def kernel(x, edge_index, edge_attr, params):
    src = edge_index[0]
    dst = edge_index[1]
    src2d = src.reshape(_NW, _NCH, _CR)
    dst2d = dst.reshape(_NW, _NCH, _CR)
    c1 = params['conv1']
    core = params['core']

    a2_c1 = jnp.stack([c1['a_src'], c1['a_dst']], axis=1)
    h1, s1 = _tc_enc(x, c1['W'], a2_c1)
    ea = _tc_ea(edge_attr, c1['We'], c1['a_edge'])
    o0, o1 = _sc_gat(h1, s1, src2d, dst2d, ea)

    prev_b = c1['b']
    zs = []
    for i in range(L):
        p = core[i]
        a2 = jnp.stack([p['a_src'], p['a_dst']], axis=1)
        z, h, s = _tc_fuse(o0, o1, prev_b, p['W'], a2)
        if i > 0:
            zs.append(z)
        o0, o1 = _sc_gat(h, s, dst2d, src2d)
        prev_b = p['b']
    ec = params['edge_cls']
    weca = ec['W1'][:H]
    wecb = ec['W1'][H:2 * H]
    node_out, pmat, qmat = _tc_jk(o0, o1, prev_b, zs[0], zs[1],
                                  params['jk'], params['node_cls'],
                                  weca, wecb, ec['b1'])
    w2d = ec['W2'][:, 1] - ec['W2'][:, 0]
    b2d = ec['b2'][1] - ec['b2'][0]
    d = _sc_ec(pmat, qmat, src2d, dst2d,
               edge_attr.reshape(_NW, _EPW * 4), ec['W1'][2 * H:], w2d)
    edge_out = _tc_ecs(d, b2d)
    return node_out, edge_out


# speedup vs baseline: 1.0328x; 1.0328x over previous
import jax
import jax.numpy as jnp
from jax import lax
from jax.experimental import pallas as pl
from jax.experimental.pallas import tpu as pltpu
from jax.experimental.pallas import tpu_sc as plsc

N = 10000
E = 320000
H = 128
L = 3
HL = (L * H) // 2
BLK = 1000


def _full(shape):
    return pl.BlockSpec(shape, lambda i: tuple(0 for _ in shape))


def _rows(shape):
    return pl.BlockSpec(shape, lambda i: (i,) + tuple(0 for _ in shape[1:]))


def _enc_body(x_ref, w_ref, a2_ref, h_ref, s_ref):
    h = jnp.dot(x_ref[...], w_ref[...], preferred_element_type=jnp.float32)
    h_ref[...] = h
    s_ref[...] = jnp.dot(h, a2_ref[...], preferred_element_type=jnp.float32)


def _tc_enc(x, w, a2):
    return pl.pallas_call(
        _enc_body,
        grid=(N // BLK,),
        in_specs=[_rows((BLK, H)), _full((H, H)), _full((H, 2))],
        out_specs=[_rows((BLK, H)), _rows((BLK, 2))],
        out_shape=[
            jax.ShapeDtypeStruct((N, H), jnp.float32),
            jax.ShapeDtypeStruct((N, 2), jnp.float32),
        ],
    )(x, w, a2)


def _phalf(shape):
    return pl.BlockSpec(shape, lambda i: (0, i, 0))


def _cat_relu(p0_ref, p1_ref, b):
    z = jnp.concatenate([p0_ref[0] + p1_ref[0],
                         p0_ref[1] + p1_ref[1]], axis=1)
    return jnp.maximum(z + b, 0.0)


def _fuse_body(p0_ref, p1_ref, b_ref, w_ref, a2_ref,
               z_ref, h_ref, s_ref):
    z = _cat_relu(p0_ref, p1_ref, b_ref[...])
    z_ref[...] = z
    h = jnp.dot(z, w_ref[...], preferred_element_type=jnp.float32)
    h_ref[...] = h
    s_ref[...] = jnp.dot(h, a2_ref[...], preferred_element_type=jnp.float32)


def _tc_fuse(p0, p1, b, w, a2):
    return pl.pallas_call(
        _fuse_body,
        grid=(N // BLK,),
        in_specs=[_phalf((2, BLK, H // 2)), _phalf((2, BLK, H // 2)),
                  _full((1, H)), _full((H, H)), _full((H, 2))],
        out_specs=[_rows((BLK, H)), _rows((BLK, H)), _rows((BLK, 2))],
        out_shape=[
            jax.ShapeDtypeStruct((N, H), jnp.float32),
            jax.ShapeDtypeStruct((N, H), jnp.float32),
            jax.ShapeDtypeStruct((N, 2), jnp.float32),
        ],
    )(p0, p1, b.reshape(1, H), w, a2)


def _lstm_steps(zs, h0, c0, w_ih, w_hh, bsum, order):
    h, c = h0, c0
    outs = [None, None, None]
    for t in order:
        g = (jnp.dot(zs[t], w_ih, preferred_element_type=jnp.float32)
             + jnp.dot(h, w_hh, preferred_element_type=jnp.float32) + bsum)
        i = jax.nn.sigmoid(g[:, 0 * HL:1 * HL])
        f = jax.nn.sigmoid(g[:, 1 * HL:2 * HL])
        gg = jnp.tanh(g[:, 2 * HL:3 * HL])
        o = jax.nn.sigmoid(g[:, 3 * HL:4 * HL])
        c = f * c + i * gg
        h = o * jnp.tanh(c)
        outs[t] = h
    return outs


def _jk_body(p0_ref, p1_ref, b_ref, z1_ref, z2_ref,
             wihf_ref, whhf_ref, bf_ref, wihb_ref, whhb_ref, bb_ref,
             attw_ref, w1_ref, b1_ref, w2_ref, b2_ref,
             weca_ref, wecb_ref, ecb1_ref,
             node_ref, p_ref, q_ref):
    z3 = _cat_relu(p0_ref, p1_ref, b_ref[...])
    zs = [z1_ref[...], z2_ref[...], z3]
    zero = jnp.zeros((zs[0].shape[0], HL), jnp.float32)
    of = _lstm_steps(zs, zero, zero, wihf_ref[...], whhf_ref[...],
                     bf_ref[...], (0, 1, 2))
    ob = _lstm_steps(zs, zero, zero, wihb_ref[...], whhb_ref[...],
                     bb_ref[...], (2, 1, 0))
    attw = attw_ref[...]
    logits = jnp.concatenate(
        [jnp.dot(jnp.concatenate([of[t], ob[t]], axis=1), attw,
                 preferred_element_type=jnp.float32) for t in range(3)],
        axis=1)
    alpha = jax.nn.softmax(logits, axis=1)
    zj = (alpha[:, 0:1] * zs[0] + alpha[:, 1:2] * zs[1]
          + alpha[:, 2:3] * zs[2])
    r = jnp.maximum(jnp.dot(zj, w1_ref[...],
                            preferred_element_type=jnp.float32) + b1_ref[...],
                    0.0)
    nl = jnp.dot(r, w2_ref[...], preferred_element_type=jnp.float32) + b2_ref[...]
    node_ref[...] = jax.nn.softmax(nl, axis=1)
    p_ref[...] = (jnp.dot(zj, weca_ref[...], preferred_element_type=jnp.float32)
                  + ecb1_ref[...])
    q_ref[...] = jnp.dot(zj, wecb_ref[...], preferred_element_type=jnp.float32)


def _tc_jk(p0, p1, b, z1, z2, jk, nc, weca, wecb, ecb1):
    bf = (jk['b_ih_f'] + jk['b_hh_f']).reshape(1, 4 * HL)
    bb = (jk['b_ih_b'] + jk['b_hh_b']).reshape(1, 4 * HL)
    return pl.pallas_call(
        _jk_body,
        grid=(N // BLK,),
        in_specs=[_phalf((2, BLK, H // 2)), _phalf((2, BLK, H // 2)),
                  _full((1, H)),
                  _rows((BLK, H)), _rows((BLK, H)),
                  _full((H, 4 * HL)), _full((HL, 4 * HL)), _full((1, 4 * HL)),
                  _full((H, 4 * HL)), _full((HL, 4 * HL)), _full((1, 4 * HL)),
                  _full((2 * HL, 1)),
                  _full((H, H)), _full((1, H)), _full((H, 2)), _full((1, 2)),
                  _full((H, H)), _full((H, H)), _full((1, H))],
        out_specs=[_rows((BLK, 2)), _rows((BLK, H)), _rows((BLK, H))],
        out_shape=[
            jax.ShapeDtypeStruct((N, 2), jnp.float32),
            jax.ShapeDtypeStruct((N, H), jnp.float32),
            jax.ShapeDtypeStruct((N, H), jnp.float32),
        ],
    )(p0, p1, b.reshape(1, H),
      z1, z2,
      jk['W_ih_f'].T, jk['W_hh_f'].T, bf,
      jk['W_ih_b'].T, jk['W_hh_b'].T, bb,
      jk['att_W'].reshape(2 * HL, 1),
      nc['W1'], nc['b1'].reshape(1, H), nc['W2'], nc['b2'].reshape(1, 2),
      weca, wecb, ecb1.reshape(1, H))


_NW = 32
_EPW = E // _NW
_CA = 2000
_CR = 80
_NCH = _EPW // _CR
_NP = 10240
_RPS = _NP // 16


def _sc_gat_body(has_ea, args):
    if has_ea:
        (h2_hbm, s_hbm, nbr3d_h, agg3d_h, ea3d_h,
         o0_hbm, o1_hbm,
         s_tab, nbr_c, agg_c, ea_c, nbr_c1, agg_c1, ea_c1,
         e_arr, nbr2d, agg2d, stab, rows, rows1,
         rowidx, idxg, idxg1, sh_ssum, sh_out,
         sem_g0, sem_s0, sem_g1, sem_s1) = args
    else:
        (h2_hbm, s_hbm, nbr3d_h, agg3d_h,
         o0_hbm, o1_hbm,
         s_tab, nbr_c, agg_c, ea_c, nbr_c1, agg_c1, ea_c1,
         e_arr, nbr2d, agg2d, stab, rows, rows1,
         rowidx, idxg, idxg1, sh_ssum, sh_out,
         sem_g0, sem_s0, sem_g1, sem_s1) = args
        ea3d_h = None
    ci = lax.axis_index("c")
    si = lax.axis_index("s")
    wid = si * 2 + ci
    zero16 = jnp.zeros((16,), jnp.float32)
    iota16 = lax.iota(jnp.int32, 16)

    def _z80(i, _):
        for kk in range(8):
            stab[i, pl.ds(kk * 16, 16)] = zero16
        return 0
    lax.fori_loop(0, 80, _z80, 0)

    def _zrows(i, _):
        for kk in range(4):
            rows[i, pl.ds(kk * 16, 16)] = zero16
        return 0
    lax.fori_loop(0, _CR, _zrows, 0)

    @pl.when(si == 0)
    def _():
        pltpu.sync_copy(stab, sh_ssum)

    for g in range(5):
        rowidx[pl.ds(g * 16, 16)] = iota16 + g * 16

    pltpu.sync_copy(s_hbm, s_tab)
    pltpu.sync_copy(nbr3d_h.at[wid], nbr2d)
    pltpu.sync_copy(agg3d_h.at[wid], agg2d)
    plsc.subcore_barrier()

    rpc = _CA // _CR
    npc = _EPW // _CA
    other = si * 2 + (1 - ci)
    pa_bufs = ((nbr_c, agg_c, ea_c, sem_g0), (nbr_c1, agg_c1, ea_c1, sem_g1))

    def _pa_issue(t, b):
        nb_b, ag_b, ea_b, sg = pa_bufs[b]
        sl = jnp.where(t < npc, wid, other)
        ch = lax.rem(t, npc)
        pltpu.async_copy(nbr3d_h.at[sl, pl.ds(ch * rpc, rpc)], nb_b, sg)
        pltpu.async_copy(agg3d_h.at[sl, pl.ds(ch * rpc, rpc)], ag_b, sg)
        if ea3d_h is not None:
            pltpu.async_copy(ea3d_h.at[sl, pl.ds(ch * rpc, rpc)], ea_b, sg)

    def _pa_wait(t, b):
        nb_b, ag_b, ea_b, sg = pa_bufs[b]
        sl = jnp.where(t < npc, wid, other)
        ch = lax.rem(t, npc)
        pltpu.make_async_copy(nbr3d_h.at[sl, pl.ds(ch * rpc, rpc)],
                              nb_b, sg).wait()
        pltpu.make_async_copy(agg3d_h.at[sl, pl.ds(ch * rpc, rpc)],
                              ag_b, sg).wait()
        if ea3d_h is not None:
            pltpu.make_async_copy(ea3d_h.at[sl, pl.ds(ch * rpc, rpc)],
                                  ea_b, sg).wait()

    def _pa_process(t, b):
        nb_b, ag_b, ea_b, sg = pa_bufs[b]
        ch = lax.rem(t, npc)

        def _r25(r, _):
            for g in range(_CR // 16):
                nb = nb_b[r, pl.ds(g * 16, 16)]
                ag = ag_b[r, pl.ds(g * 16, 16)]
                sa = plsc.load_gather(s_tab, [nb * 2])
                sb = plsc.load_gather(s_tab, [ag * 2 + 1])
                al = sa + sb
                if ea3d_h is not None:
                    al = al + ea_b[r, pl.ds(g * 16, 16)]
                al = jnp.where(al >= 0.0, al, al * 0.2)
                ev = jnp.exp(al)

                @pl.when(t < npc)
                def _(ev=ev, ch=ch, r=r, g=g):
                    e_arr[pl.ds(ch * _CA + r * _CR + g * 16, 16)] = ev
                rr = lax.shift_right_logical(ag, 7)
                cc = jnp.bitwise_and(ag, 127)
                plsc.addupdate_scatter(stab, [rr, cc], ev)
            return 0
        lax.fori_loop(0, rpc, _r25, 0)

    _pa_issue(0, 0)

    def _pa_pair(i, _):
        t0 = 2 * i
        _pa_issue(t0 + 1, 1)
        _pa_wait(t0, 0)
        _pa_process(t0, 0)

        @pl.when(t0 + 2 < 2 * npc)
        def _(t0=t0):
            _pa_issue(t0 + 2, 0)
        _pa_wait(t0 + 1, 1)
        _pa_process(t0 + 1, 1)
        return 0
    lax.fori_loop(0, npc, _pa_pair, 0)

    pltpu.sync_copy(stab, sh_ssum.at[rowidx], add=True)
    plsc.subcore_barrier()
    pltpu.sync_copy(sh_ssum, stab)

    def _wchunk(ch, _):
        for g in range(_CR // 16):
            ag = agg2d[ch, pl.ds(g * 16, 16)]
            rr = lax.shift_right_logical(ag, 7)
            cc = jnp.bitwise_and(ag, 127)
            ssum = plsc.load_gather(stab, [rr, cc])
            ev = e_arr[pl.ds(ch * _CR + g * 16, 16)]
            e_arr[pl.ds(ch * _CR + g * 16, 16)] = ev / (ssum + 1e-16)
        return 0
    lax.fori_loop(0, _NCH, _wchunk, 0)

    bufs = ((rows, idxg, sem_g0, sem_s0), (rows1, idxg1, sem_g1, sem_s1))

    def _build_idx(ch, idx_ref, half):
        for g in range(_CR // 16):
            nb = nbr2d[ch, pl.ds(g * 16, 16)]
            idx_ref[pl.ds(g * 16, 16)] = nb * 2 + half

    def _scale(ch, rows_ref):
        def _row5(i, _):
            j0 = i * 5
            wv = e_arr[pl.ds(ch * _CR + j0, 16)]
            for r5 in range(5):
                ws = wv[r5]
                for kk in range(4):
                    rows_ref[j0 + r5, pl.ds(kk * 16, 16)] = (
                        rows_ref[j0 + r5, pl.ds(kk * 16, 16)] * ws)
            return 0
        lax.fori_loop(0, _CR // 5, _row5, 0)

    for half in range(2):
        lax.fori_loop(0, _CR, _zrows, 0)
        for r in range(_RPS // _CR):
            pltpu.sync_copy(rows, sh_out.at[pl.ds(si * _RPS + r * _CR, _CR)])
        plsc.subcore_barrier()

        def _pair(i, _, half=half):
            for b, (rows_b, idx_b, sg, ss) in enumerate(bufs):
                ch = 2 * i + b

                @pl.when(i > 0)
                def _(rows_b=rows_b, ss=ss):
                    pltpu.make_async_copy(
                        rows_b, sh_out.at[agg2d.at[0]], ss).wait()
                _build_idx(ch, idx_b, half)
                pltpu.async_copy(h2_hbm.at[idx_b], rows_b, sg)
            for b, (rows_b, idx_b, sg, ss) in enumerate(bufs):
                ch = 2 * i + b
                pltpu.make_async_copy(h2_hbm.at[idx_b], rows_b, sg).wait()
                _scale(ch, rows_b)
                pltpu.async_copy(rows_b, sh_out.at[agg2d.at[ch]], ss,
                                 add=True)
            return 0
        lax.fori_loop(0, (_NCH - 1) // 2, _pair, 0)

        for rows_b, idx_b, sg, ss in bufs:
            pltpu.make_async_copy(rows_b, sh_out.at[agg2d.at[0]], ss).wait()
        last = _NCH - 1
        _build_idx(last, idxg, half)
        cp = pltpu.async_copy(h2_hbm.at[idxg], rows, sem_g0)
        cp.wait()
        _scale(last, rows)
        pltpu.sync_copy(rows, sh_out.at[agg2d.at[last]], add=True)
        plsc.subcore_barrier()

        @pl.when(ci == 0)
        def _(half=half):
            pltpu.sync_copy(sh_out.at[pl.ds(si * _RPS, _RPS)],
                            o0_hbm.at[half, pl.ds(si * _RPS, _RPS)])

        @pl.when(ci == 1)
        def _(half=half):
            pltpu.sync_copy(sh_out.at[pl.ds(si * _RPS, _RPS)],
                            o1_hbm.at[half, pl.ds(si * _RPS, _RPS)])
        plsc.subcore_barrier()


def _sc_gat(h, s2, nbr3d, agg3d, eaf=None):
    has_ea = eaf is not None
    mesh = plsc.VectorSubcoreMesh(core_axis_name="c", subcore_axis_name="s")
    scratch = [
        pltpu.VMEM((2 * N,), jnp.float32),
        pltpu.VMEM((_CA // _CR, _CR), jnp.int32),
        pltpu.VMEM((_CA // _CR, _CR), jnp.int32),
        pltpu.VMEM((_CA // _CR, _CR), jnp.float32),
        pltpu.VMEM((_CA // _CR, _CR), jnp.int32),
        pltpu.VMEM((_CA // _CR, _CR), jnp.int32),
        pltpu.VMEM((_CA // _CR, _CR), jnp.float32),
        pltpu.VMEM((_EPW + 16,), jnp.float32),
        pltpu.VMEM((_NCH, _CR), jnp.int32),
        pltpu.VMEM((_NCH, _CR), jnp.int32),
        pltpu.VMEM((80, 128), jnp.float32),
        pltpu.VMEM((_CR, H // 2), jnp.float32),
        pltpu.VMEM((_CR, H // 2), jnp.float32),
        pltpu.VMEM((80,), jnp.int32),
        pltpu.VMEM((_CR,), jnp.int32),
        pltpu.VMEM((_CR,), jnp.int32),
        pltpu.VMEM_SHARED((80, 128), jnp.float32),
        pltpu.VMEM_SHARED((_NP, H // 2), jnp.float32),
        pltpu.SemaphoreType.DMA,
        pltpu.SemaphoreType.DMA,
        pltpu.SemaphoreType.DMA,
        pltpu.SemaphoreType.DMA,
    ]
    out_type = [pltpu.HBM((2, _NP, H // 2), jnp.float32),
                pltpu.HBM((2, _NP, H // 2), jnp.float32)]
    fn = pl.kernel(lambda *args: _sc_gat_body(has_ea, args),
                   out_type=out_type, mesh=mesh, scratch_types=scratch,
                   compiler_params=pltpu.CompilerParams(
                       use_tc_tiling_on_sc=False,
                       needs_layout_passes=False))
    h2 = h.reshape(2 * N, H // 2)
    if has_ea:
        o0, o1 = fn(h2, s2.reshape(-1), nbr3d, agg3d,
                    eaf.reshape(_NW, _NCH, _CR))
    else:
        o0, o1 = fn(h2, s2.reshape(-1), nbr3d, agg3d)
    return o0, o1


def _ea_body(attr_ref, we_ref, ae_ref, out_ref):
    w4 = jnp.dot(we_ref[...], ae_ref[...], preferred_element_type=jnp.float32)
    out_ref[...] = jnp.dot(attr_ref[...], w4,
                           preferred_element_type=jnp.float32)


def _tc_ea(edge_attr, we, a_edge):
    eb = E // 16
    out = pl.pallas_call(
        _ea_body,
        grid=(16,),
        in_specs=[_rows((eb, 4)), _full((4, H)), _full((H, 1))],
        out_specs=_rows((eb, 1)),
        out_shape=jax.ShapeDtypeStruct((E, 1), jnp.float32),
    )(edge_attr, we, a_edge.reshape(H, 1))
    return out.reshape(E)


def _sc_ec_body(args):
    (p_hbm, q_hbm, src3d_h, dst3d_h, attr2d_h, w1c_hbm, w2d_hbm,
     d_hbm,
     sidx, didx, attr_c, prow, qrow, prow1, qrow1, d_arr, w1c_v, w2d_v,
     sem_p, sem_q, sem_p1, sem_q1) = args
    ci = lax.axis_index("c")
    si = lax.axis_index("s")
    wid = si * 2 + ci
    zero16 = jnp.zeros((16,), jnp.float32)
    iota16 = lax.iota(jnp.int32, 16)
    m0 = iota16 == 0
    pltpu.sync_copy(src3d_h.at[wid], sidx)
    pltpu.sync_copy(dst3d_h.at[wid], didx)
    pltpu.sync_copy(w1c_hbm, w1c_v)
    pltpu.sync_copy(w2d_hbm, w2d_v)
    bufs = ((prow, qrow, sem_p, sem_q), (prow1, qrow1, sem_p1, sem_q1))

    def _issue(ch, b):
        pr, qr, sp, sq = bufs[b]
        pltpu.async_copy(p_hbm.at[sidx.at[ch]], pr, sp)
        pltpu.async_copy(q_hbm.at[didx.at[ch]], qr, sq)

    def _process(ch, b):
        pr, qr, sp, sq = bufs[b]
        pltpu.sync_copy(attr2d_h.at[wid, pl.ds(ch * 4 * _CR, 4 * _CR)],
                        attr_c.at[pl.ds(0, 4 * _CR)])
        pltpu.make_async_copy(p_hbm.at[sidx.at[ch]], pr, sp).wait()
        pltpu.make_async_copy(q_hbm.at[didx.at[ch]], qr, sq).wait()
        ws0 = tuple(w1c_v[k, pl.ds(kk * 16, 16)]
                    for k in range(4) for kk in range(8))
        ws1 = tuple(w2d_v[pl.ds(kk * 16, 16)] for kk in range(8))

        def _row4(i, carry):
            w1c_r, w2d_r = carry
            j0 = 4 * i
            av = attr_c[pl.ds(4 * j0, 16)]
            for r4 in range(4):
                j = j0 + r4
                o = 4 * r4
                dacc = zero16
                for kk in range(8):
                    u = pr[j, pl.ds(kk * 16, 16)] + qr[j, pl.ds(kk * 16, 16)]
                    u = (u + av[o] * w1c_r[kk] + av[o + 1] * w1c_r[8 + kk]
                         + av[o + 2] * w1c_r[16 + kk]
                         + av[o + 3] * w1c_r[24 + kk])
                    u = jnp.maximum(u, 0.0)
                    dacc = dacc + u * w2d_r[kk]
                dj = jnp.sum(dacc)
                plsc.store_scatter(d_arr, [iota16 * 0 + (ch * _CR + j)],
                                   zero16 + dj, mask=m0)
            return carry
        lax.fori_loop(0, _CR // 4, _row4, (ws0, ws1))

    _issue(0, 0)

    def _pair(i, _):
        _issue(2 * i + 1, 1)
        _process(2 * i, 0)
        _issue(2 * i + 2, 0)
        _process(2 * i + 1, 1)
        return 0
    lax.fori_loop(0, (_NCH - 1) // 2, _pair, 0)
    _process(_NCH - 1, 0)
    pltpu.sync_copy(d_arr, d_hbm.at[pl.ds(wid * _EPW, _EPW)])


def _sc_ec(p, q, src3d, dst3d, attr2d, w1c, w2d):
    mesh = plsc.VectorSubcoreMesh(core_axis_name="c", subcore_axis_name="s")
    scratch = [
        pltpu.VMEM((_NCH, _CR), jnp.int32),
        pltpu.VMEM((_NCH, _CR), jnp.int32),
        pltpu.VMEM((4 * _CR + 16,), jnp.float32),
        pltpu.VMEM((_CR, H), jnp.float32),
        pltpu.VMEM((_CR, H), jnp.float32),
        pltpu.VMEM((_CR, H), jnp.float32),
        pltpu.VMEM((_CR, H), jnp.float32),
        pltpu.VMEM((_EPW,), jnp.float32),
        pltpu.VMEM((4, H), jnp.float32),
        pltpu.VMEM((H,), jnp.float32),
        pltpu.SemaphoreType.DMA,
        pltpu.SemaphoreType.DMA,
        pltpu.SemaphoreType.DMA,
        pltpu.SemaphoreType.DMA,
    ]
    fn = pl.kernel(lambda *args: _sc_ec_body(args),
                   out_type=[pltpu.HBM((E,), jnp.float32)],
                   mesh=mesh, scratch_types=scratch,
                   compiler_params=pltpu.CompilerParams(
                       use_tc_tiling_on_sc=False,
                       needs_layout_passes=False))
    (d,) = fn(p, q, src3d, dst3d, attr2d, w1c, w2d)
    return d


def _ecs_body(d_ref, b2d_ref, o0_ref, o1_ref):
    dd = d_ref[...] + b2d_ref[...]
    o0 = 1.0 / (1.0 + jnp.exp(dd))
    o0_ref[...] = o0
    o1_ref[...] = 1.0 - o0


def _tc_ecs(d, b2d):
    rows = E // 128
    o0, o1 = pl.pallas_call(
        _ecs_body,
        grid=(1,),
        in_specs=[_full((rows, 128)), _full((1, 1))],
        out_specs=[_full((rows, 128)), _full((rows, 128))],
        out_shape=[jax.ShapeDtypeStruct((rows, 128), jnp.float32),
                   jax.ShapeDtypeStruct((rows, 128), jnp.float32)],
    )(d.reshape(rows, 128), b2d.reshape(1, 1))
    return jnp.stack([o0, o1], axis=-1).reshape(E, 2)


def kernel(x, edge_index, edge_attr, params):
    src = edge_index[0]
    dst = edge_index[1]
    src2d = src.reshape(_NW, _NCH, _CR)
    dst2d = dst.reshape(_NW, _NCH, _CR)
    c1 = params['conv1']
    core = params['core']

    a2_c1 = jnp.stack([c1['a_src'], c1['a_dst']], axis=1)
    h1, s1 = _tc_enc(x, c1['W'], a2_c1)
    ea = _tc_ea(edge_attr, c1['We'], c1['a_edge'])
    o0, o1 = _sc_gat(h1, s1, src2d, dst2d, ea)

    prev_b = c1['b']
    zs = []
    for i in range(L):
        p = core[i]
        a2 = jnp.stack([p['a_src'], p['a_dst']], axis=1)
        z, h, s = _tc_fuse(o0, o1, prev_b, p['W'], a2)
        if i > 0:
            zs.append(z)
        o0, o1 = _sc_gat(h, s, dst2d, src2d)
        prev_b = p['b']
    ec = params['edge_cls']
    weca = ec['W1'][:H]
    wecb = ec['W1'][H:2 * H]
    node_out, pmat, qmat = _tc_jk(o0, o1, prev_b, zs[0], zs[1],
                                  params['jk'], params['node_cls'],
                                  weca, wecb, ec['b1'])
    w2d = ec['W2'][:, 1] - ec['W2'][:, 0]
    b2d = ec['b2'][1] - ec['b2'][0]
    d = _sc_ec(pmat, qmat, src2d, dst2d,
               edge_attr.reshape(_NW, _EPW * 4), ec['W1'][2 * H:], w2d)
    edge_out = _tc_ecs(d, b2d)
    return node_out, edge_out


# speedup vs baseline: 1.0723x; 1.0382x over previous
import jax
import jax.numpy as jnp
from jax import lax
from jax.experimental import pallas as pl
from jax.experimental.pallas import tpu as pltpu
from jax.experimental.pallas import tpu_sc as plsc

N = 10000
E = 320000
H = 128
L = 3
HL = (L * H) // 2
BLK = 1000


def _full(shape):
    return pl.BlockSpec(shape, lambda i: tuple(0 for _ in shape))


def _rows(shape):
    return pl.BlockSpec(shape, lambda i: (i,) + tuple(0 for _ in shape[1:]))


def _enc_body(x_ref, w_ref, a2_ref, h_ref, s_ref):
    h = jnp.dot(x_ref[...], w_ref[...], preferred_element_type=jnp.float32)
    h_ref[...] = h
    s_ref[...] = jnp.dot(h, a2_ref[...], preferred_element_type=jnp.float32)


def _tc_enc(x, w, a2):
    return pl.pallas_call(
        _enc_body,
        grid=(N // BLK,),
        in_specs=[_rows((BLK, H)), _full((H, H)), _full((H, 2))],
        out_specs=[_rows((BLK, H)), _rows((BLK, 2))],
        out_shape=[
            jax.ShapeDtypeStruct((N, H), jnp.float32),
            jax.ShapeDtypeStruct((N, 2), jnp.float32),
        ],
    )(x, w, a2)


def _phalf(shape):
    return pl.BlockSpec(shape, lambda i: (0, i, 0))


def _cat_relu(p0_ref, p1_ref, b):
    z = jnp.concatenate([p0_ref[0] + p1_ref[0],
                         p0_ref[1] + p1_ref[1]], axis=1)
    return jnp.maximum(z + b, 0.0)


def _fuse_body(p0_ref, p1_ref, b_ref, w_ref, a2_ref,
               z_ref, h_ref, s_ref):
    z = _cat_relu(p0_ref, p1_ref, b_ref[...])
    z_ref[...] = z
    h = jnp.dot(z, w_ref[...], preferred_element_type=jnp.float32)
    h_ref[...] = h
    s_ref[...] = jnp.dot(h, a2_ref[...], preferred_element_type=jnp.float32)


def _tc_fuse(p0, p1, b, w, a2):
    return pl.pallas_call(
        _fuse_body,
        grid=(N // BLK,),
        in_specs=[_phalf((2, BLK, H // 2)), _phalf((2, BLK, H // 2)),
                  _full((1, H)), _full((H, H)), _full((H, 2))],
        out_specs=[_rows((BLK, H)), _rows((BLK, H)), _rows((BLK, 2))],
        out_shape=[
            jax.ShapeDtypeStruct((N, H), jnp.float32),
            jax.ShapeDtypeStruct((N, H), jnp.float32),
            jax.ShapeDtypeStruct((N, 2), jnp.float32),
        ],
    )(p0, p1, b.reshape(1, H), w, a2)


def _lstm_steps(zs, h0, c0, w_ih, w_hh, bsum, order):
    h, c = h0, c0
    outs = [None, None, None]
    for t in order:
        g = (jnp.dot(zs[t], w_ih, preferred_element_type=jnp.float32)
             + jnp.dot(h, w_hh, preferred_element_type=jnp.float32) + bsum)
        i = jax.nn.sigmoid(g[:, 0 * HL:1 * HL])
        f = jax.nn.sigmoid(g[:, 1 * HL:2 * HL])
        gg = jnp.tanh(g[:, 2 * HL:3 * HL])
        o = jax.nn.sigmoid(g[:, 3 * HL:4 * HL])
        c = f * c + i * gg
        h = o * jnp.tanh(c)
        outs[t] = h
    return outs


def _jk_body(p0_ref, p1_ref, b_ref, z1_ref, z2_ref,
             wihf_ref, whhf_ref, bf_ref, wihb_ref, whhb_ref, bb_ref,
             attw_ref, w1_ref, b1_ref, w2_ref, b2_ref,
             weca_ref, wecb_ref, ecb1_ref,
             node_ref, p_ref, q_ref):
    z3 = _cat_relu(p0_ref, p1_ref, b_ref[...])
    zs = [z1_ref[...], z2_ref[...], z3]
    zero = jnp.zeros((zs[0].shape[0], HL), jnp.float32)
    of = _lstm_steps(zs, zero, zero, wihf_ref[...], whhf_ref[...],
                     bf_ref[...], (0, 1, 2))
    ob = _lstm_steps(zs, zero, zero, wihb_ref[...], whhb_ref[...],
                     bb_ref[...], (2, 1, 0))
    attw = attw_ref[...]
    logits = jnp.concatenate(
        [jnp.dot(jnp.concatenate([of[t], ob[t]], axis=1), attw,
                 preferred_element_type=jnp.float32) for t in range(3)],
        axis=1)
    alpha = jax.nn.softmax(logits, axis=1)
    zj = (alpha[:, 0:1] * zs[0] + alpha[:, 1:2] * zs[1]
          + alpha[:, 2:3] * zs[2])
    r = jnp.maximum(jnp.dot(zj, w1_ref[...],
                            preferred_element_type=jnp.float32) + b1_ref[...],
                    0.0)
    nl = jnp.dot(r, w2_ref[...], preferred_element_type=jnp.float32) + b2_ref[...]
    node_ref[...] = jax.nn.softmax(nl, axis=1)
    p_ref[...] = (jnp.dot(zj, weca_ref[...], preferred_element_type=jnp.float32)
                  + ecb1_ref[...])
    q_ref[...] = jnp.dot(zj, wecb_ref[...], preferred_element_type=jnp.float32)


def _tc_jk(p0, p1, b, z1, z2, jk, nc, weca, wecb, ecb1):
    bf = (jk['b_ih_f'] + jk['b_hh_f']).reshape(1, 4 * HL)
    bb = (jk['b_ih_b'] + jk['b_hh_b']).reshape(1, 4 * HL)
    return pl.pallas_call(
        _jk_body,
        grid=(N // BLK,),
        in_specs=[_phalf((2, BLK, H // 2)), _phalf((2, BLK, H // 2)),
                  _full((1, H)),
                  _rows((BLK, H)), _rows((BLK, H)),
                  _full((H, 4 * HL)), _full((HL, 4 * HL)), _full((1, 4 * HL)),
                  _full((H, 4 * HL)), _full((HL, 4 * HL)), _full((1, 4 * HL)),
                  _full((2 * HL, 1)),
                  _full((H, H)), _full((1, H)), _full((H, 2)), _full((1, 2)),
                  _full((H, H)), _full((H, H)), _full((1, H))],
        out_specs=[_rows((BLK, 2)), _rows((BLK, H)), _rows((BLK, H))],
        out_shape=[
            jax.ShapeDtypeStruct((N, 2), jnp.float32),
            jax.ShapeDtypeStruct((N, H), jnp.float32),
            jax.ShapeDtypeStruct((N, H), jnp.float32),
        ],
    )(p0, p1, b.reshape(1, H),
      z1, z2,
      jk['W_ih_f'].T, jk['W_hh_f'].T, bf,
      jk['W_ih_b'].T, jk['W_hh_b'].T, bb,
      jk['att_W'].reshape(2 * HL, 1),
      nc['W1'], nc['b1'].reshape(1, H), nc['W2'], nc['b2'].reshape(1, 2),
      weca, wecb, ecb1.reshape(1, H))


_NW = 32
_EPW = E // _NW
_CA = 2000
_CR = 80
_NCH = _EPW // _CR
_NP = 10240
_RPS = _NP // 16


def _sc_gat_body(has_ea, args):
    if has_ea:
        (h2_hbm, s_hbm, nbr3d_h, agg3d_h, ea3d_h,
         o0_hbm, o1_hbm,
         s_tab, nbr_c, agg_c, ea_c, nbr_c1, agg_c1, ea_c1,
         e_arr, nbr2d, agg2d, stab, rows, rows1,
         rowidx, idxg, idxg1, sh_ssum, sh_out,
         sem_g0, sem_s0, sem_g1, sem_s1) = args
    else:
        (h2_hbm, s_hbm, nbr3d_h, agg3d_h,
         o0_hbm, o1_hbm,
         s_tab, nbr_c, agg_c, ea_c, nbr_c1, agg_c1, ea_c1,
         e_arr, nbr2d, agg2d, stab, rows, rows1,
         rowidx, idxg, idxg1, sh_ssum, sh_out,
         sem_g0, sem_s0, sem_g1, sem_s1) = args
        ea3d_h = None
    ci = lax.axis_index("c")
    si = lax.axis_index("s")
    wid = si * 2 + ci
    zero16 = jnp.zeros((16,), jnp.float32)
    iota16 = lax.iota(jnp.int32, 16)

    def _z80(i, _):
        for kk in range(8):
            stab[i, pl.ds(kk * 16, 16)] = zero16
        return 0
    lax.fori_loop(0, 80, _z80, 0)

    def _zrows(i, _):
        for kk in range(4):
            rows[i, pl.ds(kk * 16, 16)] = zero16
        return 0
    lax.fori_loop(0, _CR, _zrows, 0)

    @pl.when(si == 0)
    def _():
        pltpu.sync_copy(stab, sh_ssum)

    for g in range(5):
        rowidx[pl.ds(g * 16, 16)] = iota16 + g * 16

    pltpu.sync_copy(s_hbm, s_tab)
    pltpu.sync_copy(nbr3d_h.at[wid], nbr2d)
    pltpu.sync_copy(agg3d_h.at[wid], agg2d)
    plsc.subcore_barrier()

    rpc = _CA // _CR
    npc = _EPW // _CA
    other = si * 2 + (1 - ci)
    pa_bufs = ((nbr_c, agg_c, ea_c, sem_g0), (nbr_c1, agg_c1, ea_c1, sem_g1))

    def _pa_issue(t, b):
        nb_b, ag_b, ea_b, sg = pa_bufs[b]
        sl = jnp.where(t < npc, wid, other)
        ch = lax.rem(t, npc)
        pltpu.async_copy(nbr3d_h.at[sl, pl.ds(ch * rpc, rpc)], nb_b, sg)
        pltpu.async_copy(agg3d_h.at[sl, pl.ds(ch * rpc, rpc)], ag_b, sg)
        if ea3d_h is not None:
            pltpu.async_copy(ea3d_h.at[sl, pl.ds(ch * rpc, rpc)], ea_b, sg)

    def _pa_wait(t, b):
        nb_b, ag_b, ea_b, sg = pa_bufs[b]
        sl = jnp.where(t < npc, wid, other)
        ch = lax.rem(t, npc)
        pltpu.make_async_copy(nbr3d_h.at[sl, pl.ds(ch * rpc, rpc)],
                              nb_b, sg).wait()
        pltpu.make_async_copy(agg3d_h.at[sl, pl.ds(ch * rpc, rpc)],
                              ag_b, sg).wait()
        if ea3d_h is not None:
            pltpu.make_async_copy(ea3d_h.at[sl, pl.ds(ch * rpc, rpc)],
                                  ea_b, sg).wait()

    def _pa_process(t, b):
        nb_b, ag_b, ea_b, sg = pa_bufs[b]
        ch = lax.rem(t, npc)

        def _r25(r, _):
            for g in range(_CR // 16):
                nb = nb_b[r, pl.ds(g * 16, 16)]
                ag = ag_b[r, pl.ds(g * 16, 16)]
                sa = plsc.load_gather(s_tab, [nb * 2])
                sb = plsc.load_gather(s_tab, [ag * 2 + 1])
                al = sa + sb
                if ea3d_h is not None:
                    al = al + ea_b[r, pl.ds(g * 16, 16)]
                al = jnp.where(al >= 0.0, al, al * 0.2)
                ev = jnp.exp(al)

                @pl.when(t < npc)
                def _(ev=ev, ch=ch, r=r, g=g):
                    e_arr[pl.ds(ch * _CA + r * _CR + g * 16, 16)] = ev
                rr = lax.shift_right_logical(ag, 7)
                cc = jnp.bitwise_and(ag, 127)
                plsc.addupdate_scatter(stab, [rr, cc], ev)
            return 0
        lax.fori_loop(0, rpc, _r25, 0)

    _pa_issue(0, 0)

    def _pa_pair(i, _):
        t0 = 2 * i
        _pa_issue(t0 + 1, 1)
        _pa_wait(t0, 0)
        _pa_process(t0, 0)

        @pl.when(t0 + 2 < 2 * npc)
        def _(t0=t0):
            _pa_issue(t0 + 2, 0)
        _pa_wait(t0 + 1, 1)
        _pa_process(t0 + 1, 1)
        return 0
    lax.fori_loop(0, npc, _pa_pair, 0)

    pltpu.sync_copy(stab, sh_ssum.at[rowidx], add=True)
    plsc.subcore_barrier()
    pltpu.sync_copy(sh_ssum, stab)

    def _wchunk(ch, _):
        for g in range(_CR // 16):
            ag = agg2d[ch, pl.ds(g * 16, 16)]
            rr = lax.shift_right_logical(ag, 7)
            cc = jnp.bitwise_and(ag, 127)
            ssum = plsc.load_gather(stab, [rr, cc])
            ev = e_arr[pl.ds(ch * _CR + g * 16, 16)]
            e_arr[pl.ds(ch * _CR + g * 16, 16)] = ev / (ssum + 1e-16)
        return 0
    lax.fori_loop(0, _NCH, _wchunk, 0)

    bufs = ((rows, idxg, sem_g0, sem_s0), (rows1, idxg1, sem_g1, sem_s1))

    def _build_idx(ch, idx_ref, half):
        for g in range(_CR // 16):
            nb = nbr2d[ch, pl.ds(g * 16, 16)]
            idx_ref[pl.ds(g * 16, 16)] = nb * 2 + half

    def _scale(ch, rows_ref):
        def _row5(i, _):
            j0 = i * 5
            wv = e_arr[pl.ds(ch * _CR + j0, 16)]
            for r5 in range(5):
                ws = wv[r5]
                for kk in range(4):
                    rows_ref[j0 + r5, pl.ds(kk * 16, 16)] = (
                        rows_ref[j0 + r5, pl.ds(kk * 16, 16)] * ws)
            return 0
        lax.fori_loop(0, _CR // 5, _row5, 0)

    for half in range(2):
        lax.fori_loop(0, _CR, _zrows, 0)
        for r in range(_RPS // _CR):
            pltpu.sync_copy(rows, sh_out.at[pl.ds(si * _RPS + r * _CR, _CR)])
        plsc.subcore_barrier()

        def _pair(i, _, half=half):
            for b, (rows_b, idx_b, sg, ss) in enumerate(bufs):
                ch = 2 * i + b

                @pl.when(i > 0)
                def _(rows_b=rows_b, ss=ss):
                    pltpu.make_async_copy(
                        rows_b, sh_out.at[agg2d.at[0]], ss).wait()
                _build_idx(ch, idx_b, half)
                pltpu.async_copy(h2_hbm.at[idx_b], rows_b, sg)
            for b, (rows_b, idx_b, sg, ss) in enumerate(bufs):
                ch = 2 * i + b
                pltpu.make_async_copy(h2_hbm.at[idx_b], rows_b, sg).wait()
                _scale(ch, rows_b)
                pltpu.async_copy(rows_b, sh_out.at[agg2d.at[ch]], ss,
                                 add=True)
            return 0
        lax.fori_loop(0, (_NCH - 1) // 2, _pair, 0)

        for rows_b, idx_b, sg, ss in bufs:
            pltpu.make_async_copy(rows_b, sh_out.at[agg2d.at[0]], ss).wait()
        last = _NCH - 1
        _build_idx(last, idxg, half)
        cp = pltpu.async_copy(h2_hbm.at[idxg], rows, sem_g0)
        cp.wait()
        _scale(last, rows)
        pltpu.sync_copy(rows, sh_out.at[agg2d.at[last]], add=True)
        plsc.subcore_barrier()

        @pl.when(ci == 0)
        def _(half=half):
            pltpu.sync_copy(sh_out.at[pl.ds(si * _RPS, _RPS)],
                            o0_hbm.at[half, pl.ds(si * _RPS, _RPS)])

        @pl.when(ci == 1)
        def _(half=half):
            pltpu.sync_copy(sh_out.at[pl.ds(si * _RPS, _RPS)],
                            o1_hbm.at[half, pl.ds(si * _RPS, _RPS)])
        plsc.subcore_barrier()


def _sc_gat(h, s2, nbr3d, agg3d, eaf=None):
    has_ea = eaf is not None
    mesh = plsc.VectorSubcoreMesh(core_axis_name="c", subcore_axis_name="s")
    scratch = [
        pltpu.VMEM((2 * N,), jnp.float32),
        pltpu.VMEM((_CA // _CR, _CR), jnp.int32),
        pltpu.VMEM((_CA // _CR, _CR), jnp.int32),
        pltpu.VMEM((_CA // _CR, _CR), jnp.float32),
        pltpu.VMEM((_CA // _CR, _CR), jnp.int32),
        pltpu.VMEM((_CA // _CR, _CR), jnp.int32),
        pltpu.VMEM((_CA // _CR, _CR), jnp.float32),
        pltpu.VMEM((_EPW + 16,), jnp.float32),
        pltpu.VMEM((_NCH, _CR), jnp.int32),
        pltpu.VMEM((_NCH, _CR), jnp.int32),
        pltpu.VMEM((80, 128), jnp.float32),
        pltpu.VMEM((_CR, H // 2), jnp.float32),
        pltpu.VMEM((_CR, H // 2), jnp.float32),
        pltpu.VMEM((80,), jnp.int32),
        pltpu.VMEM((_CR,), jnp.int32),
        pltpu.VMEM((_CR,), jnp.int32),
        pltpu.VMEM_SHARED((80, 128), jnp.float32),
        pltpu.VMEM_SHARED((_NP, H // 2), jnp.float32),
        pltpu.SemaphoreType.DMA,
        pltpu.SemaphoreType.DMA,
        pltpu.SemaphoreType.DMA,
        pltpu.SemaphoreType.DMA,
    ]
    out_type = [pltpu.HBM((2, _NP, H // 2), jnp.float32),
                pltpu.HBM((2, _NP, H // 2), jnp.float32)]
    fn = pl.kernel(lambda *args: _sc_gat_body(has_ea, args),
                   out_type=out_type, mesh=mesh, scratch_types=scratch,
                   compiler_params=pltpu.CompilerParams(
                       use_tc_tiling_on_sc=False,
                       needs_layout_passes=False))
    h2 = h.reshape(2 * N, H // 2)
    if has_ea:
        o0, o1 = fn(h2, s2.reshape(-1), nbr3d, agg3d,
                    eaf.reshape(_NW, _NCH, _CR))
    else:
        o0, o1 = fn(h2, s2.reshape(-1), nbr3d, agg3d)
    return o0, o1


def _ea_body(attr_ref, we_ref, ae_ref, out_ref):
    w4 = jnp.dot(we_ref[...], ae_ref[...], preferred_element_type=jnp.float32)
    out_ref[...] = jnp.dot(attr_ref[...], w4,
                           preferred_element_type=jnp.float32)


def _tc_ea(edge_attr, we, a_edge):
    eb = E // 16
    out = pl.pallas_call(
        _ea_body,
        grid=(16,),
        in_specs=[_rows((eb, 4)), _full((4, H)), _full((H, 1))],
        out_specs=_rows((eb, 1)),
        out_shape=jax.ShapeDtypeStruct((E, 1), jnp.float32),
    )(edge_attr, we, a_edge.reshape(H, 1))
    return out.reshape(E)


def _sc_ec_body(args):
    (p_hbm, q_hbm, src3d_h, dst3d_h, attr2d_h, w1c_hbm, w2d_hbm,
     d_hbm,
     sidx, didx, attr_c, prow, qrow, prow1, qrow1, d_arr, w1c_v, w2d_v,
     sem_p, sem_q, sem_p1, sem_q1) = args
    ci = lax.axis_index("c")
    si = lax.axis_index("s")
    wid = si * 2 + ci
    zero16 = jnp.zeros((16,), jnp.float32)
    iota16 = lax.iota(jnp.int32, 16)
    m0 = iota16 == 0
    pltpu.sync_copy(src3d_h.at[wid], sidx)
    pltpu.sync_copy(dst3d_h.at[wid], didx)
    pltpu.sync_copy(w1c_hbm, w1c_v)
    pltpu.sync_copy(w2d_hbm, w2d_v)
    pltpu.sync_copy(attr2d_h.at[wid], attr_c)
    bufs = ((prow, qrow, sem_p, sem_q), (prow1, qrow1, sem_p1, sem_q1))

    def _issue(ch, b):
        pr, qr, sp, sq = bufs[b]
        pltpu.async_copy(p_hbm.at[sidx.at[ch]], pr, sp)
        pltpu.async_copy(q_hbm.at[didx.at[ch]], qr, sq)

    def _process(ch, b):
        pr, qr, sp, sq = bufs[b]
        pltpu.make_async_copy(p_hbm.at[sidx.at[ch]], pr, sp).wait()
        pltpu.make_async_copy(q_hbm.at[didx.at[ch]], qr, sq).wait()
        ws0 = tuple(w1c_v[k, pl.ds(kk * 16, 16)]
                    for k in range(4) for kk in range(8))
        ws1 = tuple(w2d_v[pl.ds(kk * 16, 16)] for kk in range(8))

        def _row4(i, carry):
            w1c_r, w2d_r = carry
            j0 = 4 * i
            av = attr_c[pl.ds(ch * 4 * _CR + 4 * j0, 16)]
            for r4 in range(4):
                j = j0 + r4
                o = 4 * r4
                dacc = zero16
                for kk in range(8):
                    u = pr[j, pl.ds(kk * 16, 16)] + qr[j, pl.ds(kk * 16, 16)]
                    u = (u + av[o] * w1c_r[kk] + av[o + 1] * w1c_r[8 + kk]
                         + av[o + 2] * w1c_r[16 + kk]
                         + av[o + 3] * w1c_r[24 + kk])
                    u = jnp.maximum(u, 0.0)
                    dacc = dacc + u * w2d_r[kk]
                dj = jnp.sum(dacc)
                plsc.store_scatter(d_arr, [iota16 * 0 + (ch * _CR + j)],
                                   zero16 + dj, mask=m0)
            return carry
        lax.fori_loop(0, _CR // 4, _row4, (ws0, ws1))

    _issue(0, 0)

    def _pair(i, _):
        _issue(2 * i + 1, 1)
        _process(2 * i, 0)
        _issue(2 * i + 2, 0)
        _process(2 * i + 1, 1)
        return 0
    lax.fori_loop(0, (_NCH - 1) // 2, _pair, 0)
    _process(_NCH - 1, 0)
    pltpu.sync_copy(d_arr, d_hbm.at[pl.ds(wid * _EPW, _EPW)])


def _sc_ec(p, q, src3d, dst3d, attr2d, w1c, w2d):
    mesh = plsc.VectorSubcoreMesh(core_axis_name="c", subcore_axis_name="s")
    scratch = [
        pltpu.VMEM((_NCH, _CR), jnp.int32),
        pltpu.VMEM((_NCH, _CR), jnp.int32),
        pltpu.VMEM((4 * _EPW,), jnp.float32),
        pltpu.VMEM((_CR, H), jnp.float32),
        pltpu.VMEM((_CR, H), jnp.float32),
        pltpu.VMEM((_CR, H), jnp.float32),
        pltpu.VMEM((_CR, H), jnp.float32),
        pltpu.VMEM((_EPW,), jnp.float32),
        pltpu.VMEM((4, H), jnp.float32),
        pltpu.VMEM((H,), jnp.float32),
        pltpu.SemaphoreType.DMA,
        pltpu.SemaphoreType.DMA,
        pltpu.SemaphoreType.DMA,
        pltpu.SemaphoreType.DMA,
    ]
    fn = pl.kernel(lambda *args: _sc_ec_body(args),
                   out_type=[pltpu.HBM((E,), jnp.float32)],
                   mesh=mesh, scratch_types=scratch,
                   compiler_params=pltpu.CompilerParams(
                       use_tc_tiling_on_sc=False,
                       needs_layout_passes=False))
    (d,) = fn(p, q, src3d, dst3d, attr2d, w1c, w2d)
    return d


def _ecs_body(d_ref, b2d_ref, o0_ref, o1_ref):
    dd = d_ref[...] + b2d_ref[...]
    o0 = 1.0 / (1.0 + jnp.exp(dd))
    o0_ref[...] = o0
    o1_ref[...] = 1.0 - o0


def _tc_ecs(d, b2d):
    rows = E // 128
    o0, o1 = pl.pallas_call(
        _ecs_body,
        grid=(1,),
        in_specs=[_full((rows, 128)), _full((1, 1))],
        out_specs=[_full((rows, 128)), _full((rows, 128))],
        out_shape=[jax.ShapeDtypeStruct((rows, 128), jnp.float32),
                   jax.ShapeDtypeStruct((rows, 128), jnp.float32)],
    )(d.reshape(rows, 128), b2d.reshape(1, 1))
    return jnp.stack([o0, o1], axis=-1).reshape(E, 2)


def kernel(x, edge_index, edge_attr, params):
    src = edge_index[0]
    dst = edge_index[1]
    src2d = src.reshape(_NW, _NCH, _CR)
    dst2d = dst.reshape(_NW, _NCH, _CR)
    c1 = params['conv1']
    core = params['core']

    a2_c1 = jnp.stack([c1['a_src'], c1['a_dst']], axis=1)
    h1, s1 = _tc_enc(x, c1['W'], a2_c1)
    ea = _tc_ea(edge_attr, c1['We'], c1['a_edge'])
    o0, o1 = _sc_gat(h1, s1, src2d, dst2d, ea)

    prev_b = c1['b']
    zs = []
    for i in range(L):
        p = core[i]
        a2 = jnp.stack([p['a_src'], p['a_dst']], axis=1)
        z, h, s = _tc_fuse(o0, o1, prev_b, p['W'], a2)
        if i > 0:
            zs.append(z)
        o0, o1 = _sc_gat(h, s, dst2d, src2d)
        prev_b = p['b']
    ec = params['edge_cls']
    weca = ec['W1'][:H]
    wecb = ec['W1'][H:2 * H]
    node_out, pmat, qmat = _tc_jk(o0, o1, prev_b, zs[0], zs[1],
                                  params['jk'], params['node_cls'],
                                  weca, wecb, ec['b1'])
    w2d = ec['W2'][:, 1] - ec['W2'][:, 0]
    b2d = ec['b2'][1] - ec['b2'][0]
    d = _sc_ec(pmat, qmat, src2d, dst2d,
               edge_attr.reshape(_NW, _EPW * 4), ec['W1'][2 * H:], w2d)
    edge_out = _tc_ecs(d, b2d)
    return node_out, edge_out


# speedup vs baseline: 1.0748x; 1.0023x over previous
import jax
import jax.numpy as jnp
from jax import lax
from jax.experimental import pallas as pl
from jax.experimental.pallas import tpu as pltpu
from jax.experimental.pallas import tpu_sc as plsc

N = 10000
E = 320000
H = 128
L = 3
HL = (L * H) // 2
BLK = 1000


def _full(shape):
    return pl.BlockSpec(shape, lambda i: tuple(0 for _ in shape))


def _rows(shape):
    return pl.BlockSpec(shape, lambda i: (i,) + tuple(0 for _ in shape[1:]))


def _enc_body(x_ref, w_ref, a2_ref, h_ref, s_ref):
    h = jnp.dot(x_ref[...], w_ref[...], preferred_element_type=jnp.float32)
    h_ref[...] = h
    s_ref[...] = jnp.dot(h, a2_ref[...], preferred_element_type=jnp.float32)


def _tc_enc(x, w, a2):
    return pl.pallas_call(
        _enc_body,
        grid=(N // BLK,),
        in_specs=[_rows((BLK, H)), _full((H, H)), _full((H, 2))],
        out_specs=[_rows((BLK, H)), _rows((BLK, 2))],
        out_shape=[
            jax.ShapeDtypeStruct((N, H), jnp.float32),
            jax.ShapeDtypeStruct((N, 2), jnp.float32),
        ],
    )(x, w, a2)


def _phalf(shape):
    return pl.BlockSpec(shape, lambda i: (0, i, 0))


def _cat_relu(p0_ref, p1_ref, b):
    z = jnp.concatenate([p0_ref[0] + p1_ref[0],
                         p0_ref[1] + p1_ref[1]], axis=1)
    return jnp.maximum(z + b, 0.0)


def _fuse_body(p0_ref, p1_ref, b_ref, w_ref, a2_ref,
               z_ref, h_ref, s_ref):
    z = _cat_relu(p0_ref, p1_ref, b_ref[...])
    z_ref[...] = z
    h = jnp.dot(z, w_ref[...], preferred_element_type=jnp.float32)
    h_ref[...] = h
    s_ref[...] = jnp.dot(h, a2_ref[...], preferred_element_type=jnp.float32)


def _tc_fuse(p0, p1, b, w, a2):
    return pl.pallas_call(
        _fuse_body,
        grid=(N // BLK,),
        in_specs=[_phalf((2, BLK, H // 2)), _phalf((2, BLK, H // 2)),
                  _full((1, H)), _full((H, H)), _full((H, 2))],
        out_specs=[_rows((BLK, H)), _rows((BLK, H)), _rows((BLK, 2))],
        out_shape=[
            jax.ShapeDtypeStruct((N, H), jnp.float32),
            jax.ShapeDtypeStruct((N, H), jnp.float32),
            jax.ShapeDtypeStruct((N, 2), jnp.float32),
        ],
    )(p0, p1, b.reshape(1, H), w, a2)


def _lstm_steps(zs, h0, c0, w_ih, w_hh, bsum, order):
    h, c = h0, c0
    outs = [None, None, None]
    for t in order:
        g = (jnp.dot(zs[t], w_ih, preferred_element_type=jnp.float32)
             + jnp.dot(h, w_hh, preferred_element_type=jnp.float32) + bsum)
        i = jax.nn.sigmoid(g[:, 0 * HL:1 * HL])
        f = jax.nn.sigmoid(g[:, 1 * HL:2 * HL])
        gg = jnp.tanh(g[:, 2 * HL:3 * HL])
        o = jax.nn.sigmoid(g[:, 3 * HL:4 * HL])
        c = f * c + i * gg
        h = o * jnp.tanh(c)
        outs[t] = h
    return outs


def _jk_body(p0_ref, p1_ref, b_ref, z1_ref, z2_ref,
             wihf_ref, whhf_ref, bf_ref, wihb_ref, whhb_ref, bb_ref,
             attw_ref, w1_ref, b1_ref, w2_ref, b2_ref,
             weca_ref, wecb_ref, ecb1_ref,
             node_ref, p_ref, q_ref):
    z3 = _cat_relu(p0_ref, p1_ref, b_ref[...])
    zs = [z1_ref[...], z2_ref[...], z3]
    zero = jnp.zeros((zs[0].shape[0], HL), jnp.float32)
    of = _lstm_steps(zs, zero, zero, wihf_ref[...], whhf_ref[...],
                     bf_ref[...], (0, 1, 2))
    ob = _lstm_steps(zs, zero, zero, wihb_ref[...], whhb_ref[...],
                     bb_ref[...], (2, 1, 0))
    attw = attw_ref[...]
    logits = jnp.concatenate(
        [jnp.dot(jnp.concatenate([of[t], ob[t]], axis=1), attw,
                 preferred_element_type=jnp.float32) for t in range(3)],
        axis=1)
    alpha = jax.nn.softmax(logits, axis=1)
    zj = (alpha[:, 0:1] * zs[0] + alpha[:, 1:2] * zs[1]
          + alpha[:, 2:3] * zs[2])
    r = jnp.maximum(jnp.dot(zj, w1_ref[...],
                            preferred_element_type=jnp.float32) + b1_ref[...],
                    0.0)
    nl = jnp.dot(r, w2_ref[...], preferred_element_type=jnp.float32) + b2_ref[...]
    node_ref[...] = jax.nn.softmax(nl, axis=1)
    p_ref[...] = (jnp.dot(zj, weca_ref[...], preferred_element_type=jnp.float32)
                  + ecb1_ref[...])
    q_ref[...] = jnp.dot(zj, wecb_ref[...], preferred_element_type=jnp.float32)


def _tc_jk(p0, p1, b, z1, z2, jk, nc, weca, wecb, ecb1):
    bf = (jk['b_ih_f'] + jk['b_hh_f']).reshape(1, 4 * HL)
    bb = (jk['b_ih_b'] + jk['b_hh_b']).reshape(1, 4 * HL)
    return pl.pallas_call(
        _jk_body,
        grid=(N // BLK,),
        in_specs=[_phalf((2, BLK, H // 2)), _phalf((2, BLK, H // 2)),
                  _full((1, H)),
                  _rows((BLK, H)), _rows((BLK, H)),
                  _full((H, 4 * HL)), _full((HL, 4 * HL)), _full((1, 4 * HL)),
                  _full((H, 4 * HL)), _full((HL, 4 * HL)), _full((1, 4 * HL)),
                  _full((2 * HL, 1)),
                  _full((H, H)), _full((1, H)), _full((H, 2)), _full((1, 2)),
                  _full((H, H)), _full((H, H)), _full((1, H))],
        out_specs=[_rows((BLK, 2)), _rows((BLK, H)), _rows((BLK, H))],
        out_shape=[
            jax.ShapeDtypeStruct((N, 2), jnp.float32),
            jax.ShapeDtypeStruct((N, H), jnp.float32),
            jax.ShapeDtypeStruct((N, H), jnp.float32),
        ],
    )(p0, p1, b.reshape(1, H),
      z1, z2,
      jk['W_ih_f'].T, jk['W_hh_f'].T, bf,
      jk['W_ih_b'].T, jk['W_hh_b'].T, bb,
      jk['att_W'].reshape(2 * HL, 1),
      nc['W1'], nc['b1'].reshape(1, H), nc['W2'], nc['b2'].reshape(1, 2),
      weca, wecb, ecb1.reshape(1, H))


_NW = 32
_EPW = E // _NW
_CA = 2000
_CR = 80
_NCH = _EPW // _CR
_NP = 10240
_RPS = _NP // 16


def _sc_gat_body(has_ea, args):
    if has_ea:
        (h2_hbm, s_hbm, nbr3d_h, agg3d_h, ea3d_h,
         o0_hbm, o1_hbm,
         s_tab, nbr_c, agg_c, ea_c, nbr_c1, agg_c1, ea_c1,
         e_arr, nbr2d, agg2d, stab, rows, rows1,
         rowidx, idxg, idxg1, sh_ssum, sh_out,
         sem_g0, sem_s0, sem_g1, sem_s1) = args
    else:
        (h2_hbm, s_hbm, nbr3d_h, agg3d_h,
         o0_hbm, o1_hbm,
         s_tab, nbr_c, agg_c, ea_c, nbr_c1, agg_c1, ea_c1,
         e_arr, nbr2d, agg2d, stab, rows, rows1,
         rowidx, idxg, idxg1, sh_ssum, sh_out,
         sem_g0, sem_s0, sem_g1, sem_s1) = args
        ea3d_h = None
    ci = lax.axis_index("c")
    si = lax.axis_index("s")
    wid = si * 2 + ci
    zero16 = jnp.zeros((16,), jnp.float32)
    iota16 = lax.iota(jnp.int32, 16)

    def _z80(i, _):
        for kk in range(8):
            stab[i, pl.ds(kk * 16, 16)] = zero16
        return 0
    lax.fori_loop(0, 80, _z80, 0)

    def _zrows(i, _):
        for kk in range(4):
            rows[i, pl.ds(kk * 16, 16)] = zero16
        return 0
    lax.fori_loop(0, _CR, _zrows, 0)

    @pl.when(si == 0)
    def _():
        pltpu.sync_copy(stab, sh_ssum)

    for g in range(5):
        rowidx[pl.ds(g * 16, 16)] = iota16 + g * 16

    pltpu.sync_copy(s_hbm, s_tab)
    pltpu.sync_copy(nbr3d_h.at[wid], nbr2d)
    pltpu.sync_copy(agg3d_h.at[wid], agg2d)
    plsc.subcore_barrier()

    rpc = _CA // _CR
    npc = _EPW // _CA
    other = si * 2 + (1 - ci)
    pa_bufs = ((nbr_c, agg_c, ea_c, sem_g0), (nbr_c1, agg_c1, ea_c1, sem_g1))

    def _pa_issue(t, b):
        nb_b, ag_b, ea_b, sg = pa_bufs[b]
        sl = jnp.where(t < npc, wid, other)
        ch = lax.rem(t, npc)
        pltpu.async_copy(nbr3d_h.at[sl, pl.ds(ch * rpc, rpc)], nb_b, sg)
        pltpu.async_copy(agg3d_h.at[sl, pl.ds(ch * rpc, rpc)], ag_b, sg)
        if ea3d_h is not None:
            pltpu.async_copy(ea3d_h.at[sl, pl.ds(ch * rpc, rpc)], ea_b, sg)

    def _pa_wait(t, b):
        nb_b, ag_b, ea_b, sg = pa_bufs[b]
        sl = jnp.where(t < npc, wid, other)
        ch = lax.rem(t, npc)
        pltpu.make_async_copy(nbr3d_h.at[sl, pl.ds(ch * rpc, rpc)],
                              nb_b, sg).wait()
        pltpu.make_async_copy(agg3d_h.at[sl, pl.ds(ch * rpc, rpc)],
                              ag_b, sg).wait()
        if ea3d_h is not None:
            pltpu.make_async_copy(ea3d_h.at[sl, pl.ds(ch * rpc, rpc)],
                                  ea_b, sg).wait()

    def _pa_process(t, b):
        nb_b, ag_b, ea_b, sg = pa_bufs[b]
        ch = lax.rem(t, npc)

        def _r25(r, _):
            for g in range(_CR // 16):
                nb = nb_b[r, pl.ds(g * 16, 16)]
                ag = ag_b[r, pl.ds(g * 16, 16)]
                sa = plsc.load_gather(s_tab, [nb * 2])
                sb = plsc.load_gather(s_tab, [ag * 2 + 1])
                al = sa + sb
                if ea3d_h is not None:
                    al = al + ea_b[r, pl.ds(g * 16, 16)]
                al = jnp.where(al >= 0.0, al, al * 0.2)
                ev = jnp.exp(al)

                @pl.when(t < npc)
                def _(ev=ev, ch=ch, r=r, g=g):
                    e_arr[pl.ds(ch * _CA + r * _CR + g * 16, 16)] = ev
                rr = lax.shift_right_logical(ag, 7)
                cc = jnp.bitwise_and(ag, 127)
                plsc.addupdate_scatter(stab, [rr, cc], ev)
            return 0
        lax.fori_loop(0, rpc, _r25, 0)

    _pa_issue(0, 0)

    def _pa_pair(i, _):
        t0 = 2 * i
        _pa_issue(t0 + 1, 1)
        _pa_wait(t0, 0)
        _pa_process(t0, 0)

        @pl.when(t0 + 2 < 2 * npc)
        def _(t0=t0):
            _pa_issue(t0 + 2, 0)
        _pa_wait(t0 + 1, 1)
        _pa_process(t0 + 1, 1)
        return 0
    lax.fori_loop(0, npc, _pa_pair, 0)

    pltpu.sync_copy(stab, sh_ssum.at[rowidx], add=True)
    plsc.subcore_barrier()
    pltpu.sync_copy(sh_ssum, stab)

    def _wchunk(ch, _):
        for g in range(_CR // 16):
            ag = agg2d[ch, pl.ds(g * 16, 16)]
            rr = lax.shift_right_logical(ag, 7)
            cc = jnp.bitwise_and(ag, 127)
            ssum = plsc.load_gather(stab, [rr, cc])
            ev = e_arr[pl.ds(ch * _CR + g * 16, 16)]
            e_arr[pl.ds(ch * _CR + g * 16, 16)] = ev / (ssum + 1e-16)
        return 0
    lax.fori_loop(0, _NCH, _wchunk, 0)

    bufs = ((rows, idxg, sem_g0, sem_s0), (rows1, idxg1, sem_g1, sem_s1))

    def _build_idx(ch, idx_ref, half):
        for g in range(_CR // 16):
            nb = nbr2d[ch, pl.ds(g * 16, 16)]
            idx_ref[pl.ds(g * 16, 16)] = nb * 2 + half

    def _scale(ch, rows_ref):
        def _row5(i, _):
            j0 = i * 5
            wv = e_arr[pl.ds(ch * _CR + j0, 16)]
            for r5 in range(5):
                ws = wv[r5]
                for kk in range(4):
                    rows_ref[j0 + r5, pl.ds(kk * 16, 16)] = (
                        rows_ref[j0 + r5, pl.ds(kk * 16, 16)] * ws)
            return 0
        lax.fori_loop(0, _CR // 5, _row5, 0)

    for half in range(2):
        lax.fori_loop(0, _CR, _zrows, 0)
        for r in range(_RPS // _CR):
            pltpu.async_copy(
                rows, sh_out.at[pl.ds(si * _RPS + r * _CR, _CR)], sem_s0)
        for r in range(_RPS // _CR):
            pltpu.make_async_copy(
                rows, sh_out.at[pl.ds(si * _RPS + r * _CR, _CR)],
                sem_s0).wait()
        plsc.subcore_barrier()

        def _pair(i, _, half=half):
            for b, (rows_b, idx_b, sg, ss) in enumerate(bufs):
                ch = 2 * i + b

                @pl.when(i > 0)
                def _(rows_b=rows_b, ss=ss):
                    pltpu.make_async_copy(
                        rows_b, sh_out.at[agg2d.at[0]], ss).wait()
                _build_idx(ch, idx_b, half)
                pltpu.async_copy(h2_hbm.at[idx_b], rows_b, sg)
            for b, (rows_b, idx_b, sg, ss) in enumerate(bufs):
                ch = 2 * i + b
                pltpu.make_async_copy(h2_hbm.at[idx_b], rows_b, sg).wait()
                _scale(ch, rows_b)
                pltpu.async_copy(rows_b, sh_out.at[agg2d.at[ch]], ss,
                                 add=True)
            return 0
        lax.fori_loop(0, (_NCH - 1) // 2, _pair, 0)

        for rows_b, idx_b, sg, ss in bufs:
            pltpu.make_async_copy(rows_b, sh_out.at[agg2d.at[0]], ss).wait()
        last = _NCH - 1
        _build_idx(last, idxg, half)
        cp = pltpu.async_copy(h2_hbm.at[idxg], rows, sem_g0)
        cp.wait()
        _scale(last, rows)
        pltpu.sync_copy(rows, sh_out.at[agg2d.at[last]], add=True)
        plsc.subcore_barrier()

        @pl.when(ci == 0)
        def _(half=half):
            pltpu.sync_copy(sh_out.at[pl.ds(si * _RPS, _RPS)],
                            o0_hbm.at[half, pl.ds(si * _RPS, _RPS)])

        @pl.when(ci == 1)
        def _(half=half):
            pltpu.sync_copy(sh_out.at[pl.ds(si * _RPS, _RPS)],
                            o1_hbm.at[half, pl.ds(si * _RPS, _RPS)])
        plsc.subcore_barrier()


def _sc_gat(h, s2, nbr3d, agg3d, eaf=None):
    has_ea = eaf is not None
    mesh = plsc.VectorSubcoreMesh(core_axis_name="c", subcore_axis_name="s")
    scratch = [
        pltpu.VMEM((2 * N,), jnp.float32),
        pltpu.VMEM((_CA // _CR, _CR), jnp.int32),
        pltpu.VMEM((_CA // _CR, _CR), jnp.int32),
        pltpu.VMEM((_CA // _CR, _CR), jnp.float32),
        pltpu.VMEM((_CA // _CR, _CR), jnp.int32),
        pltpu.VMEM((_CA // _CR, _CR), jnp.int32),
        pltpu.VMEM((_CA // _CR, _CR), jnp.float32),
        pltpu.VMEM((_EPW + 16,), jnp.float32),
        pltpu.VMEM((_NCH, _CR), jnp.int32),
        pltpu.VMEM((_NCH, _CR), jnp.int32),
        pltpu.VMEM((80, 128), jnp.float32),
        pltpu.VMEM((_CR, H // 2), jnp.float32),
        pltpu.VMEM((_CR, H // 2), jnp.float32),
        pltpu.VMEM((80,), jnp.int32),
        pltpu.VMEM((_CR,), jnp.int32),
        pltpu.VMEM((_CR,), jnp.int32),
        pltpu.VMEM_SHARED((80, 128), jnp.float32),
        pltpu.VMEM_SHARED((_NP, H // 2), jnp.float32),
        pltpu.SemaphoreType.DMA,
        pltpu.SemaphoreType.DMA,
        pltpu.SemaphoreType.DMA,
        pltpu.SemaphoreType.DMA,
    ]
    out_type = [pltpu.HBM((2, _NP, H // 2), jnp.float32),
                pltpu.HBM((2, _NP, H // 2), jnp.float32)]
    fn = pl.kernel(lambda *args: _sc_gat_body(has_ea, args),
                   out_type=out_type, mesh=mesh, scratch_types=scratch,
                   compiler_params=pltpu.CompilerParams(
                       use_tc_tiling_on_sc=False,
                       needs_layout_passes=False))
    h2 = h.reshape(2 * N, H // 2)
    if has_ea:
        o0, o1 = fn(h2, s2.reshape(-1), nbr3d, agg3d,
                    eaf.reshape(_NW, _NCH, _CR))
    else:
        o0, o1 = fn(h2, s2.reshape(-1), nbr3d, agg3d)
    return o0, o1


def _ea_body(attr_ref, we_ref, ae_ref, out_ref):
    w4 = jnp.dot(we_ref[...], ae_ref[...], preferred_element_type=jnp.float32)
    out_ref[...] = jnp.dot(attr_ref[...], w4,
                           preferred_element_type=jnp.float32)


def _tc_ea(edge_attr, we, a_edge):
    eb = E // 16
    out = pl.pallas_call(
        _ea_body,
        grid=(16,),
        in_specs=[_rows((eb, 4)), _full((4, H)), _full((H, 1))],
        out_specs=_rows((eb, 1)),
        out_shape=jax.ShapeDtypeStruct((E, 1), jnp.float32),
    )(edge_attr, we, a_edge.reshape(H, 1))
    return out.reshape(E)


def _sc_ec_body(args):
    (p_hbm, q_hbm, src3d_h, dst3d_h, attr2d_h, w1c_hbm, w2d_hbm,
     d_hbm,
     sidx, didx, attr_c, prow, qrow, prow1, qrow1, d_arr, w1c_v, w2d_v,
     sem_p, sem_q, sem_p1, sem_q1) = args
    ci = lax.axis_index("c")
    si = lax.axis_index("s")
    wid = si * 2 + ci
    zero16 = jnp.zeros((16,), jnp.float32)
    iota16 = lax.iota(jnp.int32, 16)
    m0 = iota16 == 0
    pltpu.sync_copy(src3d_h.at[wid], sidx)
    pltpu.sync_copy(dst3d_h.at[wid], didx)
    pltpu.sync_copy(w1c_hbm, w1c_v)
    pltpu.sync_copy(w2d_hbm, w2d_v)
    pltpu.sync_copy(attr2d_h.at[wid], attr_c)
    bufs = ((prow, qrow, sem_p, sem_q), (prow1, qrow1, sem_p1, sem_q1))

    def _issue(ch, b):
        pr, qr, sp, sq = bufs[b]
        pltpu.async_copy(p_hbm.at[sidx.at[ch]], pr, sp)
        pltpu.async_copy(q_hbm.at[didx.at[ch]], qr, sq)

    def _process(ch, b):
        pr, qr, sp, sq = bufs[b]
        pltpu.make_async_copy(p_hbm.at[sidx.at[ch]], pr, sp).wait()
        pltpu.make_async_copy(q_hbm.at[didx.at[ch]], qr, sq).wait()
        ws0 = tuple(w1c_v[k, pl.ds(kk * 16, 16)]
                    for k in range(4) for kk in range(8))
        ws1 = tuple(w2d_v[pl.ds(kk * 16, 16)] for kk in range(8))

        def _row4(i, carry):
            w1c_r, w2d_r = carry
            j0 = 4 * i
            av = attr_c[pl.ds(ch * 4 * _CR + 4 * j0, 16)]
            for r4 in range(4):
                j = j0 + r4
                o = 4 * r4
                dacc = zero16
                for kk in range(8):
                    u = pr[j, pl.ds(kk * 16, 16)] + qr[j, pl.ds(kk * 16, 16)]
                    u = (u + av[o] * w1c_r[kk] + av[o + 1] * w1c_r[8 + kk]
                         + av[o + 2] * w1c_r[16 + kk]
                         + av[o + 3] * w1c_r[24 + kk])
                    u = jnp.maximum(u, 0.0)
                    dacc = dacc + u * w2d_r[kk]
                dj = jnp.sum(dacc)
                plsc.store_scatter(d_arr, [iota16 * 0 + (ch * _CR + j)],
                                   zero16 + dj, mask=m0)
            return carry
        lax.fori_loop(0, _CR // 4, _row4, (ws0, ws1))

    _issue(0, 0)

    def _pair(i, _):
        _issue(2 * i + 1, 1)
        _process(2 * i, 0)
        _issue(2 * i + 2, 0)
        _process(2 * i + 1, 1)
        return 0
    lax.fori_loop(0, (_NCH - 1) // 2, _pair, 0)
    _process(_NCH - 1, 0)
    pltpu.sync_copy(d_arr, d_hbm.at[pl.ds(wid * _EPW, _EPW)])


def _sc_ec(p, q, src3d, dst3d, attr2d, w1c, w2d):
    mesh = plsc.VectorSubcoreMesh(core_axis_name="c", subcore_axis_name="s")
    scratch = [
        pltpu.VMEM((_NCH, _CR), jnp.int32),
        pltpu.VMEM((_NCH, _CR), jnp.int32),
        pltpu.VMEM((4 * _EPW,), jnp.float32),
        pltpu.VMEM((_CR, H), jnp.float32),
        pltpu.VMEM((_CR, H), jnp.float32),
        pltpu.VMEM((_CR, H), jnp.float32),
        pltpu.VMEM((_CR, H), jnp.float32),
        pltpu.VMEM((_EPW,), jnp.float32),
        pltpu.VMEM((4, H), jnp.float32),
        pltpu.VMEM((H,), jnp.float32),
        pltpu.SemaphoreType.DMA,
        pltpu.SemaphoreType.DMA,
        pltpu.SemaphoreType.DMA,
        pltpu.SemaphoreType.DMA,
    ]
    fn = pl.kernel(lambda *args: _sc_ec_body(args),
                   out_type=[pltpu.HBM((E,), jnp.float32)],
                   mesh=mesh, scratch_types=scratch,
                   compiler_params=pltpu.CompilerParams(
                       use_tc_tiling_on_sc=False,
                       needs_layout_passes=False))
    (d,) = fn(p, q, src3d, dst3d, attr2d, w1c, w2d)
    return d


def _ecs_body(d_ref, b2d_ref, o0_ref, o1_ref):
    dd = d_ref[...] + b2d_ref[...]
    o0 = 1.0 / (1.0 + jnp.exp(dd))
    o0_ref[...] = o0
    o1_ref[...] = 1.0 - o0


def _tc_ecs(d, b2d):
    rows = E // 128
    o0, o1 = pl.pallas_call(
        _ecs_body,
        grid=(1,),
        in_specs=[_full((rows, 128)), _full((1, 1))],
        out_specs=[_full((rows, 128)), _full((rows, 128))],
        out_shape=[jax.ShapeDtypeStruct((rows, 128), jnp.float32),
                   jax.ShapeDtypeStruct((rows, 128), jnp.float32)],
    )(d.reshape(rows, 128), b2d.reshape(1, 1))
    return jnp.stack([o0, o1], axis=-1).reshape(E, 2)


def kernel(x, edge_index, edge_attr, params):
    src = edge_index[0]
    dst = edge_index[1]
    src2d = src.reshape(_NW, _NCH, _CR)
    dst2d = dst.reshape(_NW, _NCH, _CR)
    c1 = params['conv1']
    core = params['core']

    a2_c1 = jnp.stack([c1['a_src'], c1['a_dst']], axis=1)
    h1, s1 = _tc_enc(x, c1['W'], a2_c1)
    ea = _tc_ea(edge_attr, c1['We'], c1['a_edge'])
    o0, o1 = _sc_gat(h1, s1, src2d, dst2d, ea)

    prev_b = c1['b']
    zs = []
    for i in range(L):
        p = core[i]
        a2 = jnp.stack([p['a_src'], p['a_dst']], axis=1)
        z, h, s = _tc_fuse(o0, o1, prev_b, p['W'], a2)
        if i > 0:
            zs.append(z)
        o0, o1 = _sc_gat(h, s, dst2d, src2d)
        prev_b = p['b']
    ec = params['edge_cls']
    weca = ec['W1'][:H]
    wecb = ec['W1'][H:2 * H]
    node_out, pmat, qmat = _tc_jk(o0, o1, prev_b, zs[0], zs[1],
                                  params['jk'], params['node_cls'],
                                  weca, wecb, ec['b1'])
    w2d = ec['W2'][:, 1] - ec['W2'][:, 0]
    b2d = ec['b2'][1] - ec['b2'][0]
    d = _sc_ec(pmat, qmat, src2d, dst2d,
               edge_attr.reshape(_NW, _EPW * 4), ec['W1'][2 * H:], w2d)
    edge_out = _tc_ecs(d, b2d)
    return node_out, edge_out


# speedup vs baseline: 1.0844x; 1.0090x over previous
import jax
import jax.numpy as jnp
from jax import lax
from jax.experimental import pallas as pl
from jax.experimental.pallas import tpu as pltpu
from jax.experimental.pallas import tpu_sc as plsc

N = 10000
E = 320000
H = 128
L = 3
HL = (L * H) // 2
BLK = 1000


def _full(shape):
    return pl.BlockSpec(shape, lambda i: tuple(0 for _ in shape))


def _rows(shape):
    return pl.BlockSpec(shape, lambda i: (i,) + tuple(0 for _ in shape[1:]))


def _enc_body(x_ref, w_ref, a2_ref, h_ref, s_ref):
    h = jnp.dot(x_ref[...], w_ref[...], preferred_element_type=jnp.float32)
    h_ref[...] = h
    s_ref[...] = jnp.dot(h, a2_ref[...], preferred_element_type=jnp.float32)


def _tc_enc(x, w, a2):
    return pl.pallas_call(
        _enc_body,
        grid=(N // BLK,),
        in_specs=[_rows((BLK, H)), _full((H, H)), _full((H, 2))],
        out_specs=[_rows((BLK, H)), _rows((BLK, 2))],
        out_shape=[
            jax.ShapeDtypeStruct((N, H), jnp.float32),
            jax.ShapeDtypeStruct((N, 2), jnp.float32),
        ],
    )(x, w, a2)


def _phalf(shape):
    return pl.BlockSpec(shape, lambda i: (0, i, 0))


def _cat_relu(p0_ref, p1_ref, b):
    z = jnp.concatenate([p0_ref[0] + p1_ref[0],
                         p0_ref[1] + p1_ref[1]], axis=1)
    return jnp.maximum(z + b, 0.0)


def _fuse_body(p0_ref, p1_ref, b_ref, w_ref, a2_ref,
               z_ref, h_ref, s_ref):
    z = _cat_relu(p0_ref, p1_ref, b_ref[...])
    z_ref[...] = z
    h = jnp.dot(z, w_ref[...], preferred_element_type=jnp.float32)
    h_ref[...] = h
    s_ref[...] = jnp.dot(h, a2_ref[...], preferred_element_type=jnp.float32)


def _tc_fuse(p0, p1, b, w, a2):
    return pl.pallas_call(
        _fuse_body,
        grid=(N // BLK,),
        in_specs=[_phalf((2, BLK, H // 2)), _phalf((2, BLK, H // 2)),
                  _full((1, H)), _full((H, H)), _full((H, 2))],
        out_specs=[_rows((BLK, H)), _rows((BLK, H)), _rows((BLK, 2))],
        out_shape=[
            jax.ShapeDtypeStruct((N, H), jnp.float32),
            jax.ShapeDtypeStruct((N, H), jnp.float32),
            jax.ShapeDtypeStruct((N, 2), jnp.float32),
        ],
    )(p0, p1, b.reshape(1, H), w, a2)


def _lstm_steps(zs, h0, c0, w_ih, w_hh, bsum, order):
    h, c = h0, c0
    outs = [None, None, None]
    for t in order:
        g = (jnp.dot(zs[t], w_ih, preferred_element_type=jnp.float32)
             + jnp.dot(h, w_hh, preferred_element_type=jnp.float32) + bsum)
        i = jax.nn.sigmoid(g[:, 0 * HL:1 * HL])
        f = jax.nn.sigmoid(g[:, 1 * HL:2 * HL])
        gg = jnp.tanh(g[:, 2 * HL:3 * HL])
        o = jax.nn.sigmoid(g[:, 3 * HL:4 * HL])
        c = f * c + i * gg
        h = o * jnp.tanh(c)
        outs[t] = h
    return outs


def _jk_body(p0_ref, p1_ref, b_ref, z1_ref, z2_ref,
             wihf_ref, whhf_ref, bf_ref, wihb_ref, whhb_ref, bb_ref,
             attw_ref, w1_ref, b1_ref, w2_ref, b2_ref,
             weca_ref, wecb_ref, ecb1_ref,
             node_ref, p_ref, q_ref):
    z3 = _cat_relu(p0_ref, p1_ref, b_ref[...])
    zs = [z1_ref[...], z2_ref[...], z3]
    zero = jnp.zeros((zs[0].shape[0], HL), jnp.float32)
    of = _lstm_steps(zs, zero, zero, wihf_ref[...], whhf_ref[...],
                     bf_ref[...], (0, 1, 2))
    ob = _lstm_steps(zs, zero, zero, wihb_ref[...], whhb_ref[...],
                     bb_ref[...], (2, 1, 0))
    attw = attw_ref[...]
    logits = jnp.concatenate(
        [jnp.dot(jnp.concatenate([of[t], ob[t]], axis=1), attw,
                 preferred_element_type=jnp.float32) for t in range(3)],
        axis=1)
    alpha = jax.nn.softmax(logits, axis=1)
    zj = (alpha[:, 0:1] * zs[0] + alpha[:, 1:2] * zs[1]
          + alpha[:, 2:3] * zs[2])
    r = jnp.maximum(jnp.dot(zj, w1_ref[...],
                            preferred_element_type=jnp.float32) + b1_ref[...],
                    0.0)
    nl = jnp.dot(r, w2_ref[...], preferred_element_type=jnp.float32) + b2_ref[...]
    node_ref[...] = jax.nn.softmax(nl, axis=1)
    p_ref[...] = (jnp.dot(zj, weca_ref[...], preferred_element_type=jnp.float32)
                  + ecb1_ref[...])
    q_ref[...] = jnp.dot(zj, wecb_ref[...], preferred_element_type=jnp.float32)


def _tc_jk(p0, p1, b, z1, z2, jk, nc, weca, wecb, ecb1):
    bf = (jk['b_ih_f'] + jk['b_hh_f']).reshape(1, 4 * HL)
    bb = (jk['b_ih_b'] + jk['b_hh_b']).reshape(1, 4 * HL)
    return pl.pallas_call(
        _jk_body,
        grid=(N // BLK,),
        in_specs=[_phalf((2, BLK, H // 2)), _phalf((2, BLK, H // 2)),
                  _full((1, H)),
                  _rows((BLK, H)), _rows((BLK, H)),
                  _full((H, 4 * HL)), _full((HL, 4 * HL)), _full((1, 4 * HL)),
                  _full((H, 4 * HL)), _full((HL, 4 * HL)), _full((1, 4 * HL)),
                  _full((2 * HL, 1)),
                  _full((H, H)), _full((1, H)), _full((H, 2)), _full((1, 2)),
                  _full((H, H)), _full((H, H)), _full((1, H))],
        out_specs=[_rows((BLK, 2)), _rows((BLK, H)), _rows((BLK, H))],
        out_shape=[
            jax.ShapeDtypeStruct((N, 2), jnp.float32),
            jax.ShapeDtypeStruct((N, H), jnp.float32),
            jax.ShapeDtypeStruct((N, H), jnp.float32),
        ],
    )(p0, p1, b.reshape(1, H),
      z1, z2,
      jk['W_ih_f'].T, jk['W_hh_f'].T, bf,
      jk['W_ih_b'].T, jk['W_hh_b'].T, bb,
      jk['att_W'].reshape(2 * HL, 1),
      nc['W1'], nc['b1'].reshape(1, H), nc['W2'], nc['b2'].reshape(1, 2),
      weca, wecb, ecb1.reshape(1, H))


_NW = 32
_EPW = E // _NW
_CA = 2000
_CR = 80
_NCH = _EPW // _CR
_NP = 10240
_RPS = _NP // 16


def _sc_gat_body(has_ea, args):
    if has_ea:
        (h2_hbm, s_hbm, nbr3d_h, agg3d_h, ea3d_h,
         o0_hbm, o1_hbm,
         s_tab, nbr_c, agg_c, ea_c, nbr_c1, agg_c1, ea_c1,
         e_arr, nbr2d, agg2d, stab, rows, rows1,
         rowidx, idxg, idxg1, sh_ssum, sh_out,
         sem_g0, sem_s0, sem_g1, sem_s1) = args
    else:
        (h2_hbm, s_hbm, nbr3d_h, agg3d_h,
         o0_hbm, o1_hbm,
         s_tab, nbr_c, agg_c, ea_c, nbr_c1, agg_c1, ea_c1,
         e_arr, nbr2d, agg2d, stab, rows, rows1,
         rowidx, idxg, idxg1, sh_ssum, sh_out,
         sem_g0, sem_s0, sem_g1, sem_s1) = args
        ea3d_h = None
    ci = lax.axis_index("c")
    si = lax.axis_index("s")
    wid = si * 2 + ci
    zero16 = jnp.zeros((16,), jnp.float32)
    iota16 = lax.iota(jnp.int32, 16)

    def _z80(i, _):
        for kk in range(8):
            stab[i, pl.ds(kk * 16, 16)] = zero16
        return 0
    lax.fori_loop(0, 80, _z80, 0)

    def _zrows(i, _):
        for kk in range(4):
            rows[i, pl.ds(kk * 16, 16)] = zero16
        return 0
    lax.fori_loop(0, _CR, _zrows, 0)

    @pl.when(si == 0)
    def _():
        pltpu.sync_copy(stab, sh_ssum)

    for g in range(5):
        rowidx[pl.ds(g * 16, 16)] = iota16 + g * 16

    pltpu.sync_copy(s_hbm, s_tab)
    pltpu.sync_copy(nbr3d_h.at[wid], nbr2d)
    pltpu.sync_copy(agg3d_h.at[wid], agg2d)
    plsc.subcore_barrier()

    rpc = _CA // _CR
    npc = _EPW // _CA
    other = si * 2 + (1 - ci)
    pa_bufs = ((nbr_c, agg_c, ea_c, sem_g0), (nbr_c1, agg_c1, ea_c1, sem_g1))

    def _pa_issue(t, b):
        nb_b, ag_b, ea_b, sg = pa_bufs[b]
        sl = jnp.where(t < npc, wid, other)
        ch = lax.rem(t, npc)
        pltpu.async_copy(nbr3d_h.at[sl, pl.ds(ch * rpc, rpc)], nb_b, sg)
        pltpu.async_copy(agg3d_h.at[sl, pl.ds(ch * rpc, rpc)], ag_b, sg)
        if ea3d_h is not None:
            pltpu.async_copy(ea3d_h.at[sl, pl.ds(ch * rpc, rpc)], ea_b, sg)

    def _pa_wait(t, b):
        nb_b, ag_b, ea_b, sg = pa_bufs[b]
        sl = jnp.where(t < npc, wid, other)
        ch = lax.rem(t, npc)
        pltpu.make_async_copy(nbr3d_h.at[sl, pl.ds(ch * rpc, rpc)],
                              nb_b, sg).wait()
        pltpu.make_async_copy(agg3d_h.at[sl, pl.ds(ch * rpc, rpc)],
                              ag_b, sg).wait()
        if ea3d_h is not None:
            pltpu.make_async_copy(ea3d_h.at[sl, pl.ds(ch * rpc, rpc)],
                                  ea_b, sg).wait()

    def _pa_process(t, b):
        nb_b, ag_b, ea_b, sg = pa_bufs[b]
        ch = lax.rem(t, npc)

        def _r25(r, _):
            for g in range(_CR // 16):
                nb = nb_b[r, pl.ds(g * 16, 16)]
                ag = ag_b[r, pl.ds(g * 16, 16)]
                sa = plsc.load_gather(s_tab, [nb * 2])
                sb = plsc.load_gather(s_tab, [ag * 2 + 1])
                al = sa + sb
                if ea3d_h is not None:
                    al = al + ea_b[r, pl.ds(g * 16, 16)]
                al = jnp.where(al >= 0.0, al, al * 0.2)
                ev = jnp.exp(al)

                @pl.when(t < npc)
                def _(ev=ev, ch=ch, r=r, g=g):
                    e_arr[pl.ds(ch * _CA + r * _CR + g * 16, 16)] = ev
                rr = lax.shift_right_logical(ag, 7)
                cc = jnp.bitwise_and(ag, 127)
                plsc.addupdate_scatter(stab, [rr, cc], ev)
            return 0
        lax.fori_loop(0, rpc, _r25, 0)

    _pa_issue(0, 0)

    def _pa_pair(i, _):
        t0 = 2 * i
        _pa_issue(t0 + 1, 1)
        _pa_wait(t0, 0)
        _pa_process(t0, 0)

        @pl.when(t0 + 2 < 2 * npc)
        def _(t0=t0):
            _pa_issue(t0 + 2, 0)
        _pa_wait(t0 + 1, 1)
        _pa_process(t0 + 1, 1)
        return 0
    lax.fori_loop(0, npc, _pa_pair, 0)

    pltpu.sync_copy(stab, sh_ssum.at[rowidx], add=True)
    plsc.subcore_barrier()
    pltpu.sync_copy(sh_ssum, stab)

    def _wchunk(ch, _):
        for g in range(_CR // 16):
            ag = agg2d[ch, pl.ds(g * 16, 16)]
            rr = lax.shift_right_logical(ag, 7)
            cc = jnp.bitwise_and(ag, 127)
            ssum = plsc.load_gather(stab, [rr, cc])
            ev = e_arr[pl.ds(ch * _CR + g * 16, 16)]
            e_arr[pl.ds(ch * _CR + g * 16, 16)] = ev / (ssum + 1e-16)
        return 0
    lax.fori_loop(0, _NCH, _wchunk, 0)

    bufs = ((rows, idxg, sem_g0, sem_s0), (rows1, idxg1, sem_g1, sem_s1))

    def _build_idx(ch, idx_ref, half):
        for g in range(_CR // 16):
            nb = nbr2d[ch, pl.ds(g * 16, 16)]
            idx_ref[pl.ds(g * 16, 16)] = nb * 2 + half

    def _scale(ch, rows_ref):
        @plsc.parallel_loop(0, _CR // 5, 1, unroll=2)
        def _row5(i):
            j0 = i * 5
            wv = e_arr[pl.ds(ch * _CR + j0, 16)]
            for r5 in range(5):
                ws = wv[r5]
                for kk in range(4):
                    rows_ref[j0 + r5, pl.ds(kk * 16, 16)] = (
                        rows_ref[j0 + r5, pl.ds(kk * 16, 16)] * ws)

    for half in range(2):
        lax.fori_loop(0, _CR, _zrows, 0)
        for r in range(_RPS // _CR):
            pltpu.async_copy(
                rows, sh_out.at[pl.ds(si * _RPS + r * _CR, _CR)], sem_s0)
        for r in range(_RPS // _CR):
            pltpu.make_async_copy(
                rows, sh_out.at[pl.ds(si * _RPS + r * _CR, _CR)],
                sem_s0).wait()
        plsc.subcore_barrier()

        def _pair(i, _, half=half):
            for b, (rows_b, idx_b, sg, ss) in enumerate(bufs):
                ch = 2 * i + b

                @pl.when(i > 0)
                def _(rows_b=rows_b, ss=ss):
                    pltpu.make_async_copy(
                        rows_b, sh_out.at[agg2d.at[0]], ss).wait()
                _build_idx(ch, idx_b, half)
                pltpu.async_copy(h2_hbm.at[idx_b], rows_b, sg)
            for b, (rows_b, idx_b, sg, ss) in enumerate(bufs):
                ch = 2 * i + b
                pltpu.make_async_copy(h2_hbm.at[idx_b], rows_b, sg).wait()
                _scale(ch, rows_b)
                pltpu.async_copy(rows_b, sh_out.at[agg2d.at[ch]], ss,
                                 add=True)
            return 0
        lax.fori_loop(0, (_NCH - 1) // 2, _pair, 0)

        for rows_b, idx_b, sg, ss in bufs:
            pltpu.make_async_copy(rows_b, sh_out.at[agg2d.at[0]], ss).wait()
        last = _NCH - 1
        _build_idx(last, idxg, half)
        cp = pltpu.async_copy(h2_hbm.at[idxg], rows, sem_g0)
        cp.wait()
        _scale(last, rows)
        pltpu.sync_copy(rows, sh_out.at[agg2d.at[last]], add=True)
        plsc.subcore_barrier()

        @pl.when(ci == 0)
        def _(half=half):
            pltpu.sync_copy(sh_out.at[pl.ds(si * _RPS, _RPS)],
                            o0_hbm.at[half, pl.ds(si * _RPS, _RPS)])

        @pl.when(ci == 1)
        def _(half=half):
            pltpu.sync_copy(sh_out.at[pl.ds(si * _RPS, _RPS)],
                            o1_hbm.at[half, pl.ds(si * _RPS, _RPS)])
        plsc.subcore_barrier()


def _sc_gat(h, s2, nbr3d, agg3d, eaf=None):
    has_ea = eaf is not None
    mesh = plsc.VectorSubcoreMesh(core_axis_name="c", subcore_axis_name="s")
    scratch = [
        pltpu.VMEM((2 * N,), jnp.float32),
        pltpu.VMEM((_CA // _CR, _CR), jnp.int32),
        pltpu.VMEM((_CA // _CR, _CR), jnp.int32),
        pltpu.VMEM((_CA // _CR, _CR), jnp.float32),
        pltpu.VMEM((_CA // _CR, _CR), jnp.int32),
        pltpu.VMEM((_CA // _CR, _CR), jnp.int32),
        pltpu.VMEM((_CA // _CR, _CR), jnp.float32),
        pltpu.VMEM((_EPW + 16,), jnp.float32),
        pltpu.VMEM((_NCH, _CR), jnp.int32),
        pltpu.VMEM((_NCH, _CR), jnp.int32),
        pltpu.VMEM((80, 128), jnp.float32),
        pltpu.VMEM((_CR, H // 2), jnp.float32),
        pltpu.VMEM((_CR, H // 2), jnp.float32),
        pltpu.VMEM((80,), jnp.int32),
        pltpu.VMEM((_CR,), jnp.int32),
        pltpu.VMEM((_CR,), jnp.int32),
        pltpu.VMEM_SHARED((80, 128), jnp.float32),
        pltpu.VMEM_SHARED((_NP, H // 2), jnp.float32),
        pltpu.SemaphoreType.DMA,
        pltpu.SemaphoreType.DMA,
        pltpu.SemaphoreType.DMA,
        pltpu.SemaphoreType.DMA,
    ]
    out_type = [pltpu.HBM((2, _NP, H // 2), jnp.float32),
                pltpu.HBM((2, _NP, H // 2), jnp.float32)]
    fn = pl.kernel(lambda *args: _sc_gat_body(has_ea, args),
                   out_type=out_type, mesh=mesh, scratch_types=scratch,
                   compiler_params=pltpu.CompilerParams(
                       use_tc_tiling_on_sc=False,
                       needs_layout_passes=False))
    h2 = h.reshape(2 * N, H // 2)
    if has_ea:
        o0, o1 = fn(h2, s2.reshape(-1), nbr3d, agg3d,
                    eaf.reshape(_NW, _NCH, _CR))
    else:
        o0, o1 = fn(h2, s2.reshape(-1), nbr3d, agg3d)
    return o0, o1


def _ea_body(attr_ref, we_ref, ae_ref, out_ref):
    w4 = jnp.dot(we_ref[...], ae_ref[...], preferred_element_type=jnp.float32)
    out_ref[...] = jnp.dot(attr_ref[...], w4,
                           preferred_element_type=jnp.float32)


def _tc_ea(edge_attr, we, a_edge):
    eb = E // 16
    out = pl.pallas_call(
        _ea_body,
        grid=(16,),
        in_specs=[_rows((eb, 4)), _full((4, H)), _full((H, 1))],
        out_specs=_rows((eb, 1)),
        out_shape=jax.ShapeDtypeStruct((E, 1), jnp.float32),
    )(edge_attr, we, a_edge.reshape(H, 1))
    return out.reshape(E)


def _sc_ec_body(args):
    (p_hbm, q_hbm, src3d_h, dst3d_h, attr2d_h, w1c_hbm, w2d_hbm,
     d_hbm,
     sidx, didx, attr_c, prow, qrow, prow1, qrow1, d_arr, w1c_v, w2d_v,
     sem_p, sem_q, sem_p1, sem_q1) = args
    ci = lax.axis_index("c")
    si = lax.axis_index("s")
    wid = si * 2 + ci
    zero16 = jnp.zeros((16,), jnp.float32)
    iota16 = lax.iota(jnp.int32, 16)
    m0 = iota16 == 0
    pltpu.sync_copy(src3d_h.at[wid], sidx)
    pltpu.sync_copy(dst3d_h.at[wid], didx)
    pltpu.sync_copy(w1c_hbm, w1c_v)
    pltpu.sync_copy(w2d_hbm, w2d_v)
    pltpu.sync_copy(attr2d_h.at[wid], attr_c)
    bufs = ((prow, qrow, sem_p, sem_q), (prow1, qrow1, sem_p1, sem_q1))

    def _issue(ch, b):
        pr, qr, sp, sq = bufs[b]
        pltpu.async_copy(p_hbm.at[sidx.at[ch]], pr, sp)
        pltpu.async_copy(q_hbm.at[didx.at[ch]], qr, sq)

    def _process(ch, b):
        pr, qr, sp, sq = bufs[b]
        pltpu.make_async_copy(p_hbm.at[sidx.at[ch]], pr, sp).wait()
        pltpu.make_async_copy(q_hbm.at[didx.at[ch]], qr, sq).wait()
        ws0 = tuple(w1c_v[k, pl.ds(kk * 16, 16)]
                    for k in range(4) for kk in range(8))
        ws1 = tuple(w2d_v[pl.ds(kk * 16, 16)] for kk in range(8))

        def _row4(i, carry):
            w1c_r, w2d_r = carry
            j0 = 4 * i
            av = attr_c[pl.ds(ch * 4 * _CR + 4 * j0, 16)]
            for r4 in range(4):
                j = j0 + r4
                o = 4 * r4
                dacc = zero16
                for kk in range(8):
                    u = pr[j, pl.ds(kk * 16, 16)] + qr[j, pl.ds(kk * 16, 16)]
                    u = (u + av[o] * w1c_r[kk] + av[o + 1] * w1c_r[8 + kk]
                         + av[o + 2] * w1c_r[16 + kk]
                         + av[o + 3] * w1c_r[24 + kk])
                    u = jnp.maximum(u, 0.0)
                    dacc = dacc + u * w2d_r[kk]
                dj = jnp.sum(dacc)
                plsc.store_scatter(d_arr, [iota16 * 0 + (ch * _CR + j)],
                                   zero16 + dj, mask=m0)
            return carry
        lax.fori_loop(0, _CR // 4, _row4, (ws0, ws1))

    _issue(0, 0)

    def _pair(i, _):
        _issue(2 * i + 1, 1)
        _process(2 * i, 0)
        _issue(2 * i + 2, 0)
        _process(2 * i + 1, 1)
        return 0
    lax.fori_loop(0, (_NCH - 1) // 2, _pair, 0)
    _process(_NCH - 1, 0)
    pltpu.sync_copy(d_arr, d_hbm.at[pl.ds(wid * _EPW, _EPW)])


def _sc_ec(p, q, src3d, dst3d, attr2d, w1c, w2d):
    mesh = plsc.VectorSubcoreMesh(core_axis_name="c", subcore_axis_name="s")
    scratch = [
        pltpu.VMEM((_NCH, _CR), jnp.int32),
        pltpu.VMEM((_NCH, _CR), jnp.int32),
        pltpu.VMEM((4 * _EPW,), jnp.float32),
        pltpu.VMEM((_CR, H), jnp.float32),
        pltpu.VMEM((_CR, H), jnp.float32),
        pltpu.VMEM((_CR, H), jnp.float32),
        pltpu.VMEM((_CR, H), jnp.float32),
        pltpu.VMEM((_EPW,), jnp.float32),
        pltpu.VMEM((4, H), jnp.float32),
        pltpu.VMEM((H,), jnp.float32),
        pltpu.SemaphoreType.DMA,
        pltpu.SemaphoreType.DMA,
        pltpu.SemaphoreType.DMA,
        pltpu.SemaphoreType.DMA,
    ]
    fn = pl.kernel(lambda *args: _sc_ec_body(args),
                   out_type=[pltpu.HBM((E,), jnp.float32)],
                   mesh=mesh, scratch_types=scratch,
                   compiler_params=pltpu.CompilerParams(
                       use_tc_tiling_on_sc=False,
                       needs_layout_passes=False))
    (d,) = fn(p, q, src3d, dst3d, attr2d, w1c, w2d)
    return d


def _ecs_body(d_ref, b2d_ref, o0_ref, o1_ref):
    dd = d_ref[...] + b2d_ref[...]
    o0 = 1.0 / (1.0 + jnp.exp(dd))
    o0_ref[...] = o0
    o1_ref[...] = 1.0 - o0


def _tc_ecs(d, b2d):
    rows = E // 128
    o0, o1 = pl.pallas_call(
        _ecs_body,
        grid=(1,),
        in_specs=[_full((rows, 128)), _full((1, 1))],
        out_specs=[_full((rows, 128)), _full((rows, 128))],
        out_shape=[jax.ShapeDtypeStruct((rows, 128), jnp.float32),
                   jax.ShapeDtypeStruct((rows, 128), jnp.float32)],
    )(d.reshape(rows, 128), b2d.reshape(1, 1))
    return jnp.stack([o0, o1], axis=-1).reshape(E, 2)


def kernel(x, edge_index, edge_attr, params):
    src = edge_index[0]
    dst = edge_index[1]
    src2d = src.reshape(_NW, _NCH, _CR)
    dst2d = dst.reshape(_NW, _NCH, _CR)
    c1 = params['conv1']
    core = params['core']

    a2_c1 = jnp.stack([c1['a_src'], c1['a_dst']], axis=1)
    h1, s1 = _tc_enc(x, c1['W'], a2_c1)
    ea = _tc_ea(edge_attr, c1['We'], c1['a_edge'])
    o0, o1 = _sc_gat(h1, s1, src2d, dst2d, ea)

    prev_b = c1['b']
    zs = []
    for i in range(L):
        p = core[i]
        a2 = jnp.stack([p['a_src'], p['a_dst']], axis=1)
        z, h, s = _tc_fuse(o0, o1, prev_b, p['W'], a2)
        if i > 0:
            zs.append(z)
        o0, o1 = _sc_gat(h, s, dst2d, src2d)
        prev_b = p['b']
    ec = params['edge_cls']
    weca = ec['W1'][:H]
    wecb = ec['W1'][H:2 * H]
    node_out, pmat, qmat = _tc_jk(o0, o1, prev_b, zs[0], zs[1],
                                  params['jk'], params['node_cls'],
                                  weca, wecb, ec['b1'])
    w2d = ec['W2'][:, 1] - ec['W2'][:, 0]
    b2d = ec['b2'][1] - ec['b2'][0]
    d = _sc_ec(pmat, qmat, src2d, dst2d,
               edge_attr.reshape(_NW, _EPW * 4), ec['W1'][2 * H:], w2d)
    edge_out = _tc_ecs(d, b2d)
    return node_out, edge_out
